# Initial kernel scaffold; baseline (speedup 1.0000x reference)
import jax
import jax.numpy as jnp
from jax.experimental import pallas as pl


def kernel(x, edge_index, coords, edge_attr, We1, be1, We2, be2, Wc1, bc1, Wc2, Wn1, bn1, Wn2, bn2):
    raise NotImplementedError("write your pallas kernel here")


# speedup vs baseline: 2.1520x; 2.1520x over previous
import functools

import jax
import jax.numpy as jnp
from jax import lax
from jax.experimental import pallas as pl
from jax.experimental.pallas import tpu as pltpu
from jax.experimental.pallas import tpu_sc as plsc

N = 10000
E = 320000
D = 128
ED = 16

NC = 2
NS = 16
NW = NC * NS
TB = 128
BPT = 80
EP = NW * TB * BPT
NP = 10240
RPT = NP // NS

_f32 = jnp.float32
_i32 = jnp.int32


def _pre_body(x_ref, wr_ref, wc_ref, xr_ref, xc_ref):
    xb = x_ref[...]
    xr_ref[...] = jnp.dot(xb, wr_ref[...], preferred_element_type=_f32)
    xc_ref[...] = jnp.dot(xb, wc_ref[...], preferred_element_type=_f32)


def _node_pre(x, We1_r, We1_c):
    nb = 10
    bs = N // nb
    return pl.pallas_call(
        _pre_body,
        grid=(nb,),
        in_specs=[
            pl.BlockSpec((bs, D), lambda i: (i, 0)),
            pl.BlockSpec((D, D), lambda i: (0, 0)),
            pl.BlockSpec((D, D), lambda i: (0, 0)),
        ],
        out_specs=[
            pl.BlockSpec((bs, D), lambda i: (i, 0)),
            pl.BlockSpec((bs, D), lambda i: (i, 0)),
        ],
        out_shape=[
            jax.ShapeDtypeStruct((N, D), _f32),
            jax.ShapeDtypeStruct((N, D), _f32),
        ],
    )(x, We1_r, We1_c)


def _gather_body(xr1, xc1, c16, rowg, colg, g1o, g2o, cro, cco,
                 idxr, idxc, g1, g2, cr, cc, sem):
    c = lax.axis_index("c")
    s = lax.axis_index("s")
    wid = s * NC + c
    pltpu.sync_copy(rowg.at[pl.ds(wid * BPT, BPT)], idxr)
    pltpu.sync_copy(colg.at[pl.ds(wid * BPT, BPT)], idxc)

    @pl.loop(0, BPT)
    def _batch(j):
        base = pl.multiple_of(wid * (BPT * TB) + j * TB, TB)
        d1 = pltpu.async_copy(xr1.at[idxr.at[j]], g1, sem)
        d2 = pltpu.async_copy(xc1.at[idxc.at[j]], g2, sem)
        d3 = pltpu.async_copy(c16.at[idxr.at[j]], cr, sem)
        d4 = pltpu.async_copy(c16.at[idxc.at[j]], cc, sem)
        d1.wait()
        d2.wait()
        d3.wait()
        d4.wait()
        pltpu.sync_copy(g1, g1o.at[pl.ds(base, TB)])
        pltpu.sync_copy(g2, g2o.at[pl.ds(base, TB)])
        pltpu.sync_copy(cr, cro.at[pl.ds(base, TB)])
        pltpu.sync_copy(cc, cco.at[pl.ds(base, TB)])


def _edge_gather(xr1, xc1, c16, rowg, colg):
    mesh = plsc.VectorSubcoreMesh(core_axis_name="c", subcore_axis_name="s")
    fn = pl.kernel(
        _gather_body,
        out_type=[
            jax.ShapeDtypeStruct((EP, D), _f32),
            jax.ShapeDtypeStruct((EP, D), _f32),
            jax.ShapeDtypeStruct((EP, ED), _f32),
            jax.ShapeDtypeStruct((EP, ED), _f32),
        ],
        mesh=mesh,
        scratch_types=[
            pltpu.VMEM((BPT, TB), _i32),
            pltpu.VMEM((BPT, TB), _i32),
            pltpu.VMEM((TB, D), _f32),
            pltpu.VMEM((TB, D), _f32),
            pltpu.VMEM((TB, ED), _f32),
            pltpu.VMEM((TB, ED), _f32),
            pltpu.SemaphoreType.DMA,
        ],
        compiler_params=pltpu.CompilerParams(use_tc_tiling_on_sc=False),
    )
    return fn(xr1, xc1, c16, rowg, colg)


def _edge_body(g1, g2, cr, cc, ea, we1e, wd, be1, we2, be2, wc1, bc1, wc2,
               ef_o, cu_o):
    diff = cr[...] - cc[...]
    dist = jnp.sum(diff * diff, axis=1, keepdims=True)
    pre = (g1[...] + g2[...]
           + jnp.dot(ea[...], we1e[...], preferred_element_type=_f32)
           + dist * wd[...] + be1[...])
    h = pre * jax.nn.sigmoid(pre)
    ef = jnp.dot(h, we2[...], preferred_element_type=_f32) + be2[...]
    ef_o[...] = ef
    cv = jnp.dot(ef, wc1[...], preferred_element_type=_f32) + bc1[...]
    cs = cv * jax.nn.sigmoid(cv)
    sc = jnp.dot(cs, wc2[...], preferred_element_type=_f32)
    cu_o[...] = diff * (sc / (jnp.sqrt(dist) + 1e-8))


def _edge_mlp(g1, g2, cr, cc, ea, we1e, wd, be1, we2, be2, wc1, bc1, wc2):
    bs = 512
    nb = EP // bs
    full = lambda r, c: pl.BlockSpec((r, c), lambda i: (0, 0))
    return pl.pallas_call(
        _edge_body,
        grid=(nb,),
        in_specs=[
            pl.BlockSpec((bs, D), lambda i: (i, 0)),
            pl.BlockSpec((bs, D), lambda i: (i, 0)),
            pl.BlockSpec((bs, ED), lambda i: (i, 0)),
            pl.BlockSpec((bs, ED), lambda i: (i, 0)),
            pl.BlockSpec((bs, ED), lambda i: (i, 0)),
            full(ED, D), full(1, D), full(1, D), full(D, D), full(1, D),
            full(D, D), full(1, D), full(D, 1),
        ],
        out_specs=[
            pl.BlockSpec((bs, D), lambda i: (i, 0)),
            pl.BlockSpec((bs, ED), lambda i: (i, 0)),
        ],
        out_shape=[
            jax.ShapeDtypeStruct((EP, D), _f32),
            jax.ShapeDtypeStruct((EP, ED), _f32),
        ],
    )(g1, g2, cr, cc, ea, we1e, wd, be1, we2, be2, wc1, bc1, wc2)


def _scatter_body(efh, cuh, rowsg, z128, z16, aggo, cago,
                  idx, ef, cu, acc, acc16):
    c = lax.axis_index("c")
    s = lax.axis_index("s")
    wid = s * NC + c
    pltpu.sync_copy(z128, acc.at[pl.ds(s * RPT, RPT)])
    pltpu.sync_copy(z16, acc16.at[pl.ds(s * RPT, RPT)])
    pltpu.sync_copy(rowsg.at[pl.ds(wid * BPT, BPT)], idx)
    plsc.subcore_barrier()

    @pl.loop(0, BPT)
    def _batch(j):
        base = pl.multiple_of(wid * (BPT * TB) + j * TB, TB)
        pltpu.sync_copy(efh.at[pl.ds(base, TB)], ef)
        pltpu.sync_copy(cuh.at[pl.ds(base, TB)], cu)
        pltpu.sync_copy(ef, acc.at[idx.at[j]], add=True)
        pltpu.sync_copy(cu, acc16.at[idx.at[j]], add=True)

    plsc.subcore_barrier()
    pltpu.sync_copy(acc.at[pl.ds(s * RPT, RPT)],
                    aggo.at[c].at[pl.ds(s * RPT, RPT)])
    pltpu.sync_copy(acc16.at[pl.ds(s * RPT, RPT)],
                    cago.at[c].at[pl.ds(s * RPT, RPT)])


def _scatter(efh, cuh, rowsg, z128, z16):
    mesh = plsc.VectorSubcoreMesh(core_axis_name="c", subcore_axis_name="s")
    fn = pl.kernel(
        _scatter_body,
        out_type=[
            jax.ShapeDtypeStruct((NC, NP, D), _f32),
            jax.ShapeDtypeStruct((NC, NP, ED), _f32),
        ],
        mesh=mesh,
        scratch_types=[
            pltpu.VMEM((BPT, TB), _i32),
            pltpu.VMEM((TB, D), _f32),
            pltpu.VMEM((TB, ED), _f32),
            pltpu.VMEM_SHARED((NP, D), _f32),
            pltpu.VMEM_SHARED((NP, ED), _f32),
        ],
        compiler_params=pltpu.CompilerParams(use_tc_tiling_on_sc=False),
    )
    return fn(efh, cuh, rowsg, z128, z16)


def _node_body(x, a0, a1, cg0, cg1, c16, wn1x, wn1a, bn1, wn2, bn2,
               xn_o, cn_o):
    agg = a0[...] + a1[...]
    t = (jnp.dot(x[...], wn1x[...], preferred_element_type=_f32)
         + jnp.dot(agg, wn1a[...], preferred_element_type=_f32) + bn1[...])
    nmid = t * jax.nn.sigmoid(t)
    xn_o[...] = jnp.dot(nmid, wn2[...], preferred_element_type=_f32) + bn2[...]
    cn_o[...] = c16[...] + cg0[...] + cg1[...]


def _node_mlp(x, a0, a1, cg0, cg1, c16, wn1x, wn1a, bn1, wn2, bn2):
    nb = 10
    bs = N // nb
    full = lambda r, c: pl.BlockSpec((r, c), lambda i: (0, 0))
    return pl.pallas_call(
        _node_body,
        grid=(nb,),
        in_specs=[
            pl.BlockSpec((bs, D), lambda i: (i, 0)),
            pl.BlockSpec((bs, D), lambda i: (i, 0)),
            pl.BlockSpec((bs, D), lambda i: (i, 0)),
            pl.BlockSpec((bs, ED), lambda i: (i, 0)),
            pl.BlockSpec((bs, ED), lambda i: (i, 0)),
            pl.BlockSpec((bs, ED), lambda i: (i, 0)),
            full(D, D), full(D, D), full(1, D), full(D, D), full(1, D),
        ],
        out_specs=[
            pl.BlockSpec((bs, D), lambda i: (i, 0)),
            pl.BlockSpec((bs, ED), lambda i: (i, 0)),
        ],
        out_shape=[
            jax.ShapeDtypeStruct((N, D), _f32),
            jax.ShapeDtypeStruct((N, ED), _f32),
        ],
    )(x, a0, a1, cg0, cg1, c16, wn1x, wn1a, bn1, wn2, bn2)


def kernel(x, edge_index, coords, edge_attr,
           We1, be1, We2, be2, Wc1, bc1, Wc2, Wn1, bn1, Wn2, bn2):
    row = edge_index[0].astype(_i32)
    col = edge_index[1].astype(_i32)
    pad = EP - E
    rowg = jnp.concatenate([row, jnp.zeros((pad,), _i32)]).reshape(EP // TB, TB)
    colg = jnp.concatenate([col, jnp.zeros((pad,), _i32)]).reshape(EP // TB, TB)
    rowsg = jnp.concatenate([row, jnp.full((pad,), N, _i32)]).reshape(EP // TB, TB)
    c16 = jnp.pad(coords, ((0, 0), (0, ED - 3)))
    eap = jnp.pad(edge_attr, ((0, pad), (0, 0)))

    xr1, xc1 = _node_pre(x, We1[:D], We1[D:2 * D])
    g1, g2, cr, cc = _edge_gather(xr1, xc1, c16, rowg, colg)

    we1e = We1[2 * D:2 * D + ED]
    wd = We1[2 * D + ED:]
    ef, cu = _edge_mlp(g1, g2, cr, cc, eap,
                       we1e, wd, be1.reshape(1, D), We2, be2.reshape(1, D),
                       Wc1, bc1.reshape(1, D), Wc2)

    z128 = jnp.zeros((RPT, D), _f32)
    z16 = jnp.zeros((RPT, ED), _f32)
    agg, cag = _scatter(ef, cu, rowsg, z128, z16)

    xn, cn = _node_mlp(x, agg[0, :N], agg[1, :N], cag[0, :N], cag[1, :N],
                       c16, Wn1[:D], Wn1[D:], bn1.reshape(1, D),
                       Wn2, bn2.reshape(1, D))
    return (xn, cn[:, :3])


# speedup vs baseline: 2.4527x; 1.1398x over previous
import functools

import jax
import jax.numpy as jnp
from jax import lax
from jax.experimental import pallas as pl
from jax.experimental.pallas import tpu as pltpu
from jax.experimental.pallas import tpu_sc as plsc

N = 10000
E = 320000
D = 128
ED = 16

NC = 2
NS = 16
NW = NC * NS
TB = 128
BPT = 80
EP = NW * TB * BPT
NP = 10240
RPT = NP // NS

_f32 = jnp.float32
_i32 = jnp.int32


def _pre_body(x_ref, wr_ref, wc_ref, xr_ref, xc_ref):
    xb = x_ref[...]
    xr_ref[...] = jnp.dot(xb, wr_ref[...], preferred_element_type=_f32)
    xc_ref[...] = jnp.dot(xb, wc_ref[...], preferred_element_type=_f32)


def _node_pre(x, We1_r, We1_c):
    nb = 10
    bs = N // nb
    return pl.pallas_call(
        _pre_body,
        grid=(nb,),
        in_specs=[
            pl.BlockSpec((bs, D), lambda i: (i, 0)),
            pl.BlockSpec((D, D), lambda i: (0, 0)),
            pl.BlockSpec((D, D), lambda i: (0, 0)),
        ],
        out_specs=[
            pl.BlockSpec((bs, D), lambda i: (i, 0)),
            pl.BlockSpec((bs, D), lambda i: (i, 0)),
        ],
        out_shape=[
            jax.ShapeDtypeStruct((N, D), _f32),
            jax.ShapeDtypeStruct((N, D), _f32),
        ],
    )(x, We1_r, We1_c)


def _gather_body(xr1, xc1, c16, rowg, colg, g1o, g2o, cro, cco,
                 idxr, idxc, g1, g2, cr, cc, gs0, gs1, ws0, ws1):
    c = lax.axis_index("c")
    s = lax.axis_index("s")
    wid = s * NC + c
    gsem = (gs0, gs1)
    wsem = (ws0, ws1)
    pltpu.sync_copy(rowg.at[pl.ds(wid * BPT, BPT)], idxr)
    pltpu.sync_copy(colg.at[pl.ds(wid * BPT, BPT)], idxc)

    def gfire(j, b):
        pltpu.async_copy(xr1.at[idxr.at[j]], g1.at[b], gsem[b])
        pltpu.async_copy(xc1.at[idxc.at[j]], g2.at[b], gsem[b])
        pltpu.async_copy(c16.at[idxr.at[j]], cr.at[b], gsem[b])
        pltpu.async_copy(c16.at[idxc.at[j]], cc.at[b], gsem[b])

    def gdrain(b):
        pltpu.make_async_copy(xr1.at[pl.ds(0, TB)], g1.at[b], gsem[b]).wait()
        pltpu.make_async_copy(xc1.at[pl.ds(0, TB)], g2.at[b], gsem[b]).wait()
        pltpu.make_async_copy(c16.at[pl.ds(0, TB)], cr.at[b], gsem[b]).wait()
        pltpu.make_async_copy(c16.at[pl.ds(0, TB)], cc.at[b], gsem[b]).wait()

    def wfire(j, b):
        base = pl.multiple_of(wid * (BPT * TB) + j * TB, TB)
        pltpu.async_copy(g1.at[b], g1o.at[pl.ds(base, TB)], wsem[b])
        pltpu.async_copy(g2.at[b], g2o.at[pl.ds(base, TB)], wsem[b])
        pltpu.async_copy(cr.at[b], cro.at[pl.ds(base, TB)], wsem[b])
        pltpu.async_copy(cc.at[b], cco.at[pl.ds(base, TB)], wsem[b])

    def wdrain(b):
        pltpu.make_async_copy(g1.at[b], g1o.at[pl.ds(0, TB)], wsem[b]).wait()
        pltpu.make_async_copy(g2.at[b], g2o.at[pl.ds(0, TB)], wsem[b]).wait()
        pltpu.make_async_copy(cr.at[b], cro.at[pl.ds(0, TB)], wsem[b]).wait()
        pltpu.make_async_copy(cc.at[b], cco.at[pl.ds(0, TB)], wsem[b]).wait()

    gfire(0, 0)
    gfire(1, 1)

    @pl.loop(0, BPT - 2, step=2)
    def _batch(j):
        for b in range(2):
            jj = j + b
            gdrain(b)
            wfire(jj, b)
            wdrain(b)
            gfire(jj + 2, b)

    for b in range(2):
        gdrain(b)
        wfire(BPT - 2 + b, b)
        wdrain(b)


def _edge_gather(xr1, xc1, c16, rowg, colg):
    mesh = plsc.VectorSubcoreMesh(core_axis_name="c", subcore_axis_name="s")
    fn = pl.kernel(
        _gather_body,
        out_type=[
            jax.ShapeDtypeStruct((EP, D), _f32),
            jax.ShapeDtypeStruct((EP, D), _f32),
            jax.ShapeDtypeStruct((EP, ED), _f32),
            jax.ShapeDtypeStruct((EP, ED), _f32),
        ],
        mesh=mesh,
        scratch_types=[
            pltpu.VMEM((BPT, TB), _i32),
            pltpu.VMEM((BPT, TB), _i32),
            pltpu.VMEM((2, TB, D), _f32),
            pltpu.VMEM((2, TB, D), _f32),
            pltpu.VMEM((2, TB, ED), _f32),
            pltpu.VMEM((2, TB, ED), _f32),
            pltpu.SemaphoreType.DMA,
            pltpu.SemaphoreType.DMA,
            pltpu.SemaphoreType.DMA,
            pltpu.SemaphoreType.DMA,
        ],
        compiler_params=pltpu.CompilerParams(use_tc_tiling_on_sc=False),
    )
    return fn(xr1, xc1, c16, rowg, colg)


def _edge_body(g1, g2, cr, cc, ea, we1e, wd, be1, we2, be2, wc1, bc1, wc2,
               ef_o, cu_o):
    diff = cr[...] - cc[...]
    dist = jnp.sum(diff * diff, axis=1, keepdims=True)
    pre = (g1[...] + g2[...]
           + jnp.dot(ea[...], we1e[...], preferred_element_type=_f32)
           + dist * wd[...] + be1[...])
    h = pre * jax.nn.sigmoid(pre)
    ef = jnp.dot(h, we2[...], preferred_element_type=_f32) + be2[...]
    ef_o[...] = ef
    cv = jnp.dot(ef, wc1[...], preferred_element_type=_f32) + bc1[...]
    cs = cv * jax.nn.sigmoid(cv)
    sc = jnp.dot(cs, wc2[...], preferred_element_type=_f32)
    cu_o[...] = diff * (sc / (jnp.sqrt(dist) + 1e-8))


def _edge_mlp(g1, g2, cr, cc, ea, we1e, wd, be1, we2, be2, wc1, bc1, wc2):
    bs = 512
    nb = EP // bs
    full = lambda r, c: pl.BlockSpec((r, c), lambda i: (0, 0))
    return pl.pallas_call(
        _edge_body,
        grid=(nb,),
        in_specs=[
            pl.BlockSpec((bs, D), lambda i: (i, 0)),
            pl.BlockSpec((bs, D), lambda i: (i, 0)),
            pl.BlockSpec((bs, ED), lambda i: (i, 0)),
            pl.BlockSpec((bs, ED), lambda i: (i, 0)),
            pl.BlockSpec((bs, ED), lambda i: (i, 0)),
            full(ED, D), full(1, D), full(1, D), full(D, D), full(1, D),
            full(D, D), full(1, D), full(D, 1),
        ],
        out_specs=[
            pl.BlockSpec((bs, D), lambda i: (i, 0)),
            pl.BlockSpec((bs, ED), lambda i: (i, 0)),
        ],
        out_shape=[
            jax.ShapeDtypeStruct((EP, D), _f32),
            jax.ShapeDtypeStruct((EP, ED), _f32),
        ],
    )(g1, g2, cr, cc, ea, we1e, wd, be1, we2, be2, wc1, bc1, wc2)


def _make_scatter_body(width):
    def body(efh, rowsg, zrows, aggo, idx, ef, acc, rs0, rs1):
        c = lax.axis_index("c")
        s = lax.axis_index("s")
        wid = s * NC + c
        pltpu.sync_copy(zrows, acc.at[pl.ds(s * RPT, RPT)])
        pltpu.sync_copy(rowsg.at[pl.ds(wid * BPT, BPT)], idx)
        plsc.subcore_barrier()
        rsem = (rs0, rs1)

        def rfire(j, b):
            base = pl.multiple_of(wid * (BPT * TB) + j * TB, TB)
            pltpu.async_copy(efh.at[pl.ds(base, TB)], ef.at[b], rsem[b])

        def rdrain(b):
            pltpu.make_async_copy(efh.at[pl.ds(0, TB)], ef.at[b],
                                  rsem[b]).wait()

        rfire(0, 0)
        rfire(1, 1)

        @pl.loop(0, BPT - 2, step=2)
        def _batch(j):
            for b in range(2):
                jj = j + b
                rdrain(b)
                pltpu.sync_copy(ef.at[b], acc.at[idx.at[jj]], add=True)
                rfire(jj + 2, b)

        for b in range(2):
            jj = BPT - 2 + b
            rdrain(b)
            pltpu.sync_copy(ef.at[b], acc.at[idx.at[jj]], add=True)

        plsc.subcore_barrier()
        pltpu.sync_copy(acc.at[pl.ds(s * RPT, RPT)],
                        aggo.at[c].at[pl.ds(s * RPT, RPT)])

    return body


def _scatter(efh, rowsg, zrows, width):
    mesh = plsc.VectorSubcoreMesh(core_axis_name="c", subcore_axis_name="s")
    fn = pl.kernel(
        _make_scatter_body(width),
        out_type=jax.ShapeDtypeStruct((NC, NP, width), _f32),
        mesh=mesh,
        scratch_types=[
            pltpu.VMEM((BPT, TB), _i32),
            pltpu.VMEM((2, TB, width), _f32),
            pltpu.VMEM_SHARED((NP, width), _f32),
            pltpu.SemaphoreType.DMA,
            pltpu.SemaphoreType.DMA,
        ],
        compiler_params=pltpu.CompilerParams(use_tc_tiling_on_sc=False),
    )
    return fn(efh, rowsg, zrows)


def _node_body(x, a0, a1, cg0, cg1, c16, wn1x, wn1a, bn1, wn2, bn2,
               xn_o, cn_o):
    agg = a0[...] + a1[...]
    t = (jnp.dot(x[...], wn1x[...], preferred_element_type=_f32)
         + jnp.dot(agg, wn1a[...], preferred_element_type=_f32) + bn1[...])
    nmid = t * jax.nn.sigmoid(t)
    xn_o[...] = jnp.dot(nmid, wn2[...], preferred_element_type=_f32) + bn2[...]
    cn_o[...] = c16[...] + cg0[...] + cg1[...]


def _node_mlp(x, a0, a1, cg0, cg1, c16, wn1x, wn1a, bn1, wn2, bn2):
    nb = 10
    bs = N // nb
    full = lambda r, c: pl.BlockSpec((r, c), lambda i: (0, 0))
    return pl.pallas_call(
        _node_body,
        grid=(nb,),
        in_specs=[
            pl.BlockSpec((bs, D), lambda i: (i, 0)),
            pl.BlockSpec((bs, D), lambda i: (i, 0)),
            pl.BlockSpec((bs, D), lambda i: (i, 0)),
            pl.BlockSpec((bs, ED), lambda i: (i, 0)),
            pl.BlockSpec((bs, ED), lambda i: (i, 0)),
            pl.BlockSpec((bs, ED), lambda i: (i, 0)),
            full(D, D), full(D, D), full(1, D), full(D, D), full(1, D),
        ],
        out_specs=[
            pl.BlockSpec((bs, D), lambda i: (i, 0)),
            pl.BlockSpec((bs, ED), lambda i: (i, 0)),
        ],
        out_shape=[
            jax.ShapeDtypeStruct((N, D), _f32),
            jax.ShapeDtypeStruct((N, ED), _f32),
        ],
    )(x, a0, a1, cg0, cg1, c16, wn1x, wn1a, bn1, wn2, bn2)


def kernel(x, edge_index, coords, edge_attr,
           We1, be1, We2, be2, Wc1, bc1, Wc2, Wn1, bn1, Wn2, bn2):
    row = edge_index[0].astype(_i32)
    col = edge_index[1].astype(_i32)
    pad = EP - E
    rowg = jnp.concatenate([row, jnp.zeros((pad,), _i32)]).reshape(EP // TB, TB)
    colg = jnp.concatenate([col, jnp.zeros((pad,), _i32)]).reshape(EP // TB, TB)
    rowsg = jnp.concatenate([row, jnp.full((pad,), N, _i32)]).reshape(EP // TB, TB)
    c16 = jnp.pad(coords, ((0, 0), (0, ED - 3)))
    eap = jnp.pad(edge_attr, ((0, pad), (0, 0)))

    xr1, xc1 = _node_pre(x, We1[:D], We1[D:2 * D])
    g1, g2, cr, cc = _edge_gather(xr1, xc1, c16, rowg, colg)

    we1e = We1[2 * D:2 * D + ED]
    wd = We1[2 * D + ED:]
    ef, cu = _edge_mlp(g1, g2, cr, cc, eap,
                       we1e, wd, be1.reshape(1, D), We2, be2.reshape(1, D),
                       Wc1, bc1.reshape(1, D), Wc2)

    z128 = jnp.zeros((RPT, D), _f32)
    z16 = jnp.zeros((RPT, ED), _f32)
    agg = _scatter(ef, rowsg, z128, D)
    cag = _scatter(cu, rowsg, z16, ED)

    xn, cn = _node_mlp(x, agg[0, :N], agg[1, :N], cag[0, :N], cag[1, :N],
                       c16, Wn1[:D], Wn1[D:], bn1.reshape(1, D),
                       Wn2, bn2.reshape(1, D))
    return (xn, cn[:, :3])


# speedup vs baseline: 3.6069x; 1.4705x over previous
import jax
import jax.numpy as jnp
from jax import lax
from jax.experimental import pallas as pl
from jax.experimental.pallas import tpu as pltpu
from jax.experimental.pallas import tpu_sc as plsc

N = 10000
E = 320000
D = 128
ED = 16

NC = 2
NS = 16
TB = 80
NBT = E // TB
B0 = 168
B1 = 82
NB0T = NS * B0
BS = NBT // (NC * NS)
NP = 10240
RPT = NP // NS

_f32 = jnp.float32
_i32 = jnp.int32

_SC_PARAMS = pltpu.CompilerParams(use_tc_tiling_on_sc=False)


def _pre_body(x_ref, wr_ref, wc_ref, xr_ref, xc_ref):
    xb = x_ref[...]
    xr_ref[...] = jnp.dot(xb, wr_ref[...], preferred_element_type=_f32)
    xc_ref[...] = jnp.dot(xb, wc_ref[...], preferred_element_type=_f32)


def _node_pre(x, We1_r, We1_c):
    nb = 10
    bs = N // nb
    return pl.pallas_call(
        _pre_body,
        grid=(nb,),
        in_specs=[
            pl.BlockSpec((bs, D), lambda i: (i, 0)),
            pl.BlockSpec((D, D), lambda i: (0, 0)),
            pl.BlockSpec((D, D), lambda i: (0, 0)),
        ],
        out_specs=[
            pl.BlockSpec((bs, D), lambda i: (i, 0)),
            pl.BlockSpec((bs, D), lambda i: (i, 0)),
        ],
        out_shape=[
            jax.ShapeDtypeStruct((N, D), _f32),
            jax.ShapeDtypeStruct((N, D), _f32),
        ],
    )(x, We1_r, We1_c)


def _gather_body(xr1, xc1, c16, rowg, colg, g1o, g2o, crco,
                 idxr, idxc, g1, g2, cr, cc, gs0, gs1, ws0, ws1):
    c = lax.axis_index("c")
    s = lax.axis_index("s")
    gsem = (gs0, gs1)
    wsem = (ws0, ws1)
    nb = jnp.where(c == 0, B0, B1)
    off = jnp.where(c == 0, s * B0, NB0T + s * B1)
    pltpu.sync_copy(rowg.at[pl.ds(off, B1)], idxr.at[pl.ds(0, B1)])
    pltpu.sync_copy(colg.at[pl.ds(off, B1)], idxc.at[pl.ds(0, B1)])

    @pl.when(c == 0)
    def _rest():
        pltpu.sync_copy(rowg.at[pl.ds(off + B1, B0 - B1)],
                        idxr.at[pl.ds(B1, B0 - B1)])
        pltpu.sync_copy(colg.at[pl.ds(off + B1, B0 - B1)],
                        idxc.at[pl.ds(B1, B0 - B1)])

    def gfire(j, b):
        pltpu.async_copy(xr1.at[idxr.at[j]], g1.at[b], gsem[b])
        pltpu.async_copy(xc1.at[idxc.at[j]], g2.at[b], gsem[b])
        pltpu.async_copy(c16.at[idxr.at[j]], cr.at[b], gsem[b])
        pltpu.async_copy(c16.at[idxc.at[j]], cc.at[b], gsem[b])

    def gdrain(b):
        pltpu.make_async_copy(xr1.at[pl.ds(0, TB)], g1.at[b], gsem[b]).wait()
        pltpu.make_async_copy(xc1.at[pl.ds(0, TB)], g2.at[b], gsem[b]).wait()
        pltpu.make_async_copy(c16.at[pl.ds(0, TB)], cr.at[b], gsem[b]).wait()
        pltpu.make_async_copy(c16.at[pl.ds(0, TB)], cc.at[b], gsem[b]).wait()

    def wfire(j, b):
        base = pl.multiple_of((off + j) * TB, TB)
        pltpu.async_copy(g1.at[b], g1o.at[pl.ds(base, TB)], wsem[b])
        pltpu.async_copy(g2.at[b], g2o.at[pl.ds(base, TB)], wsem[b])
        pltpu.async_copy(cr.at[b], crco.at[pl.ds(base, TB), pl.ds(0, ED)],
                         wsem[b])
        pltpu.async_copy(cc.at[b], crco.at[pl.ds(base, TB), pl.ds(ED, ED)],
                         wsem[b])

    def wdrain(b):
        pltpu.make_async_copy(g1.at[b], g1o.at[pl.ds(0, TB)], wsem[b]).wait()
        pltpu.make_async_copy(g2.at[b], g2o.at[pl.ds(0, TB)], wsem[b]).wait()
        pltpu.make_async_copy(cr.at[b], crco.at[pl.ds(0, TB), pl.ds(0, ED)],
                              wsem[b]).wait()
        pltpu.make_async_copy(cc.at[b], crco.at[pl.ds(0, TB), pl.ds(ED, ED)],
                              wsem[b]).wait()

    gfire(0, 0)
    gfire(1, 1)

    @pl.loop(0, nb - 2, step=2)
    def _batch(j):
        for b in range(2):
            jj = j + b
            gdrain(b)
            wfire(jj, b)
            wdrain(b)
            gfire(jj + 2, b)

    for b in range(2):
        gdrain(b)
        wfire(nb - 2 + b, b)
        wdrain(b)


def _edge_gather(xr1, xc1, c16, rowg, colg):
    mesh = plsc.VectorSubcoreMesh(core_axis_name="c", subcore_axis_name="s")
    fn = pl.kernel(
        _gather_body,
        out_type=[
            jax.ShapeDtypeStruct((E, D), _f32),
            jax.ShapeDtypeStruct((E, D), _f32),
            jax.ShapeDtypeStruct((E, D), _f32),
        ],
        mesh=mesh,
        scratch_types=[
            pltpu.VMEM((B0, TB), _i32),
            pltpu.VMEM((B0, TB), _i32),
            pltpu.VMEM((2, TB, D), _f32),
            pltpu.VMEM((2, TB, D), _f32),
            pltpu.VMEM((2, TB, ED), _f32),
            pltpu.VMEM((2, TB, ED), _f32),
            pltpu.SemaphoreType.DMA,
            pltpu.SemaphoreType.DMA,
            pltpu.SemaphoreType.DMA,
            pltpu.SemaphoreType.DMA,
        ],
        compiler_params=_SC_PARAMS,
    )
    return fn(xr1, xc1, c16, rowg, colg)


def _edge_body(g1, g2, crc, ea, we1e, wd, be1, we2, be2, wc1, bc1, wc2,
               ef_o, cu_o):
    crcv = crc[...]
    diff = crcv[:, 0:ED] - crcv[:, ED:2 * ED]
    dist = jnp.sum(diff * diff, axis=1, keepdims=True)
    pre = (g1[...] + g2[...]
           + jnp.dot(ea[...], we1e[...], preferred_element_type=_f32)
           + dist * wd[...] + be1[...])
    h = pre * jax.nn.sigmoid(pre)
    ef = jnp.dot(h, we2[...], preferred_element_type=_f32) + be2[...]
    ef_o[...] = ef
    cv = jnp.dot(ef, wc1[...], preferred_element_type=_f32) + bc1[...]
    cs = cv * jax.nn.sigmoid(cv)
    sc = jnp.dot(cs, wc2[...], preferred_element_type=_f32)
    cu = diff * (sc / (jnp.sqrt(dist) + 1e-8))
    cu_o[...] = jnp.concatenate(
        [cu, jnp.zeros((cu.shape[0], D - ED), _f32)], axis=1)


def _edge_mlp(g1, g2, crc, ea, we1e, wd, be1, we2, be2, wc1, bc1, wc2):
    bs = 512
    nb = E // bs
    full = lambda r, c: pl.BlockSpec((r, c), lambda i: (0, 0))
    return pl.pallas_call(
        _edge_body,
        grid=(nb,),
        in_specs=[
            pl.BlockSpec((bs, D), lambda i: (i, 0)),
            pl.BlockSpec((bs, D), lambda i: (i, 0)),
            pl.BlockSpec((bs, D), lambda i: (i, 0)),
            pl.BlockSpec((bs, ED), lambda i: (i, 0)),
            full(ED, D), full(1, D), full(1, D), full(D, D), full(1, D),
            full(D, D), full(1, D), full(D, 1),
        ],
        out_specs=[
            pl.BlockSpec((bs, D), lambda i: (i, 0)),
            pl.BlockSpec((bs, D), lambda i: (i, 0)),
        ],
        out_shape=[
            jax.ShapeDtypeStruct((E, D), _f32),
            jax.ShapeDtypeStruct((E, D), _f32),
        ],
    )(g1, g2, crc, ea, we1e, wd, be1, we2, be2, wc1, bc1, wc2)


def _make_scatter_body(width, lanes):
    def body(efh, rowg, zrows, aggo, idx, ef, acc, rs0, rs1):
        c = lax.axis_index("c")
        s = lax.axis_index("s")
        wid = s * NC + c
        pltpu.sync_copy(zrows, acc.at[pl.ds(s * RPT, RPT)])
        pltpu.sync_copy(rowg.at[pl.ds(wid * BS, BS)], idx.at[pl.ds(0, BS)])
        plsc.subcore_barrier()
        rsem = (rs0, rs1)

        def src(j):
            base = pl.multiple_of((wid * BS + j) * TB, TB)
            if lanes == width:
                return efh.at[pl.ds(base, TB)]
            return efh.at[pl.ds(base, TB), pl.ds(0, lanes)]

        def src0():
            if lanes == width:
                return efh.at[pl.ds(0, TB)]
            return efh.at[pl.ds(0, TB), pl.ds(0, lanes)]

        def rfire(j, b):
            pltpu.async_copy(src(j), ef.at[b], rsem[b])

        def rdrain(b):
            pltpu.make_async_copy(src0(), ef.at[b], rsem[b]).wait()

        rfire(0, 0)
        rfire(1, 1)

        @pl.loop(0, BS - 3, step=2)
        def _batch(j):
            for b in range(2):
                jj = j + b
                rdrain(b)
                pltpu.sync_copy(ef.at[b], acc.at[idx.at[jj]], add=True)
                rfire(jj + 2, b)

        for b in range(2):
            jj = BS - 3 + b
            rdrain(b)
            pltpu.sync_copy(ef.at[b], acc.at[idx.at[jj]], add=True)
            if b == 0:
                rfire(BS - 1, 0)
        rdrain(0)
        pltpu.sync_copy(ef.at[0], acc.at[idx.at[BS - 1]], add=True)

        plsc.subcore_barrier()
        pltpu.sync_copy(acc.at[pl.ds(s * RPT, RPT)],
                        aggo.at[c].at[pl.ds(s * RPT, RPT)])

    return body


def _scatter(efh, rowg, zrows, width, lanes):
    mesh = plsc.VectorSubcoreMesh(core_axis_name="c", subcore_axis_name="s")
    fn = pl.kernel(
        _make_scatter_body(width, lanes),
        out_type=jax.ShapeDtypeStruct((NC, NP, lanes), _f32),
        mesh=mesh,
        scratch_types=[
            pltpu.VMEM((BS, TB), _i32),
            pltpu.VMEM((2, TB, lanes), _f32),
            pltpu.VMEM_SHARED((NP, lanes), _f32),
            pltpu.SemaphoreType.DMA,
            pltpu.SemaphoreType.DMA,
        ],
        compiler_params=_SC_PARAMS,
    )
    return fn(efh, rowg, zrows)


def _node_body(x, a0, a1, cg0, cg1, c16, wn1x, wn1a, bn1, wn2, bn2,
               xn_o, cn_o):
    agg = a0[...] + a1[...]
    t = (jnp.dot(x[...], wn1x[...], preferred_element_type=_f32)
         + jnp.dot(agg, wn1a[...], preferred_element_type=_f32) + bn1[...])
    nmid = t * jax.nn.sigmoid(t)
    xn_o[...] = jnp.dot(nmid, wn2[...], preferred_element_type=_f32) + bn2[...]
    cn_o[...] = c16[...] + cg0[...] + cg1[...]


def _node_mlp(x, a0, a1, cg0, cg1, c16, wn1x, wn1a, bn1, wn2, bn2):
    nb = 10
    bs = N // nb
    full = lambda r, c: pl.BlockSpec((r, c), lambda i: (0, 0))
    return pl.pallas_call(
        _node_body,
        grid=(nb,),
        in_specs=[
            pl.BlockSpec((bs, D), lambda i: (i, 0)),
            pl.BlockSpec((bs, D), lambda i: (i, 0)),
            pl.BlockSpec((bs, D), lambda i: (i, 0)),
            pl.BlockSpec((bs, ED), lambda i: (i, 0)),
            pl.BlockSpec((bs, ED), lambda i: (i, 0)),
            pl.BlockSpec((bs, ED), lambda i: (i, 0)),
            full(D, D), full(D, D), full(1, D), full(D, D), full(1, D),
        ],
        out_specs=[
            pl.BlockSpec((bs, D), lambda i: (i, 0)),
            pl.BlockSpec((bs, ED), lambda i: (i, 0)),
        ],
        out_shape=[
            jax.ShapeDtypeStruct((N, D), _f32),
            jax.ShapeDtypeStruct((N, ED), _f32),
        ],
    )(x, a0, a1, cg0, cg1, c16, wn1x, wn1a, bn1, wn2, bn2)


def kernel(x, edge_index, coords, edge_attr,
           We1, be1, We2, be2, Wc1, bc1, Wc2, Wn1, bn1, Wn2, bn2):
    row = edge_index[0].astype(_i32)
    col = edge_index[1].astype(_i32)
    rowg = row.reshape(NBT, TB)
    colg = col.reshape(NBT, TB)
    c16 = jnp.pad(coords, ((0, 0), (0, ED - 3)))

    xr1, xc1 = _node_pre(x, We1[:D], We1[D:2 * D])
    g1, g2, crc = _edge_gather(xr1, xc1, c16, rowg, colg)

    we1e = We1[2 * D:2 * D + ED]
    wd = We1[2 * D + ED:]
    ef, cu = _edge_mlp(g1, g2, crc, edge_attr,
                       we1e, wd, be1.reshape(1, D), We2, be2.reshape(1, D),
                       Wc1, bc1.reshape(1, D), Wc2)

    z128 = jnp.zeros((RPT, D), _f32)
    z16 = jnp.zeros((RPT, ED), _f32)
    agg = _scatter(ef, rowg, z128, D, D)
    cag = _scatter(cu, rowg, z16, D, ED)

    xn, cn = _node_mlp(x, agg[0, :N], agg[1, :N], cag[0, :N], cag[1, :N],
                       c16, Wn1[:D], Wn1[D:], bn1.reshape(1, D),
                       Wn2, bn2.reshape(1, D))
    return (xn, cn[:, :3])


# speedup vs baseline: 3.8876x; 1.0778x over previous
import jax
import jax.numpy as jnp
from jax import lax
from jax.experimental import pallas as pl
from jax.experimental.pallas import tpu as pltpu
from jax.experimental.pallas import tpu_sc as plsc

N = 10000
E = 320000
D = 128
ED = 16

NC = 2
NS = 16
TB = 80
NBT = E // TB
B0 = 168
B1 = 82
NB0T = NS * B0
BS = NBT // (NC * NS)
NP = 10240
RPT = NP // NS

_f32 = jnp.float32
_i32 = jnp.int32

_SC_PARAMS = pltpu.CompilerParams(use_tc_tiling_on_sc=False)


def _pre_body(x_ref, wr_ref, wc_ref, xr_ref, xc_ref):
    xb = x_ref[...]
    xr_ref[...] = jnp.dot(xb, wr_ref[...], preferred_element_type=_f32)
    xc_ref[...] = jnp.dot(xb, wc_ref[...], preferred_element_type=_f32)


def _node_pre(x, We1_r, We1_c):
    nb = 10
    bs = N // nb
    return pl.pallas_call(
        _pre_body,
        grid=(nb,),
        in_specs=[
            pl.BlockSpec((bs, D), lambda i: (i, 0)),
            pl.BlockSpec((D, D), lambda i: (0, 0)),
            pl.BlockSpec((D, D), lambda i: (0, 0)),
        ],
        out_specs=[
            pl.BlockSpec((bs, D), lambda i: (i, 0)),
            pl.BlockSpec((bs, D), lambda i: (i, 0)),
        ],
        out_shape=[
            jax.ShapeDtypeStruct((N, D), _f32),
            jax.ShapeDtypeStruct((N, D), _f32),
        ],
    )(x, We1_r, We1_c)


def _gather_body(xr1, xc1, c16, rowg, colg, g1o, g2o, crco,
                 idxr, idxc, g1, g2, cr, cc, gs0, gs1, ws0, ws1):
    c = lax.axis_index("c")
    s = lax.axis_index("s")
    gsem = (gs0, gs1)
    wsem = (ws0, ws1)
    nb = jnp.where(c == 0, B0, B1)
    off = jnp.where(c == 0, s * B0, NB0T + s * B1)
    pltpu.sync_copy(rowg.at[pl.ds(off, B1)], idxr.at[pl.ds(0, B1)])
    pltpu.sync_copy(colg.at[pl.ds(off, B1)], idxc.at[pl.ds(0, B1)])

    @pl.when(c == 0)
    def _rest():
        pltpu.sync_copy(rowg.at[pl.ds(off + B1, B0 - B1)],
                        idxr.at[pl.ds(B1, B0 - B1)])
        pltpu.sync_copy(colg.at[pl.ds(off + B1, B0 - B1)],
                        idxc.at[pl.ds(B1, B0 - B1)])

    def gfire(j, b):
        pltpu.async_copy(xr1.at[idxr.at[j]], g1.at[b], gsem[b])
        pltpu.async_copy(xc1.at[idxc.at[j]], g2.at[b], gsem[b])
        pltpu.async_copy(c16.at[idxr.at[j]], cr.at[b], gsem[b])
        pltpu.async_copy(c16.at[idxc.at[j]], cc.at[b], gsem[b])

    def gdrain(b):
        pltpu.make_async_copy(xr1.at[pl.ds(0, TB)], g1.at[b], gsem[b]).wait()
        pltpu.make_async_copy(xc1.at[pl.ds(0, TB)], g2.at[b], gsem[b]).wait()
        pltpu.make_async_copy(c16.at[pl.ds(0, TB)], cr.at[b], gsem[b]).wait()
        pltpu.make_async_copy(c16.at[pl.ds(0, TB)], cc.at[b], gsem[b]).wait()

    def wfire(j, b):
        base = pl.multiple_of((off + j) * TB, TB)
        pltpu.async_copy(g1.at[b], g1o.at[pl.ds(base, TB)], wsem[b])
        pltpu.async_copy(g2.at[b], g2o.at[pl.ds(base, TB)], wsem[b])
        pltpu.async_copy(cr.at[b], crco.at[pl.ds(base, TB), pl.ds(0, ED)],
                         wsem[b])
        pltpu.async_copy(cc.at[b], crco.at[pl.ds(base, TB), pl.ds(ED, ED)],
                         wsem[b])

    def wdrain(b):
        pltpu.make_async_copy(g1.at[b], g1o.at[pl.ds(0, TB)], wsem[b]).wait()
        pltpu.make_async_copy(g2.at[b], g2o.at[pl.ds(0, TB)], wsem[b]).wait()
        pltpu.make_async_copy(cr.at[b], crco.at[pl.ds(0, TB), pl.ds(0, ED)],
                              wsem[b]).wait()
        pltpu.make_async_copy(cc.at[b], crco.at[pl.ds(0, TB), pl.ds(ED, ED)],
                              wsem[b]).wait()

    gfire(0, 0)
    gfire(1, 1)

    @pl.loop(0, nb - 2, step=2)
    def _batch(j):
        for b in range(2):
            jj = j + b
            gdrain(b)
            wfire(jj, b)
            wdrain(b)
            gfire(jj + 2, b)

    for b in range(2):
        gdrain(b)
        wfire(nb - 2 + b, b)
        wdrain(b)


def _edge_gather(xr1, xc1, c16, rowg, colg):
    mesh = plsc.VectorSubcoreMesh(core_axis_name="c", subcore_axis_name="s")
    fn = pl.kernel(
        _gather_body,
        out_type=[
            jax.ShapeDtypeStruct((E, D), _f32),
            jax.ShapeDtypeStruct((E, D), _f32),
            jax.ShapeDtypeStruct((E, D), _f32),
        ],
        mesh=mesh,
        scratch_types=[
            pltpu.VMEM((B0, TB), _i32),
            pltpu.VMEM((B0, TB), _i32),
            pltpu.VMEM((2, TB, D), _f32),
            pltpu.VMEM((2, TB, D), _f32),
            pltpu.VMEM((2, TB, ED), _f32),
            pltpu.VMEM((2, TB, ED), _f32),
            pltpu.SemaphoreType.DMA,
            pltpu.SemaphoreType.DMA,
            pltpu.SemaphoreType.DMA,
            pltpu.SemaphoreType.DMA,
        ],
        compiler_params=_SC_PARAMS,
    )
    return fn(xr1, xc1, c16, rowg, colg)


def _edge_body(g1, g2, crc, eat, we1e, wd, be1, we2, be2, wc1, bc1, wc2,
               ef_o, cu_o):
    crcv = crc[...]
    diff = crcv[:, 0:ED] - crcv[:, ED:2 * ED]
    dist = jnp.sum(diff * diff, axis=1, keepdims=True)
    eaterm = lax.dot_general(eat[...], we1e[...], (((0,), (0,)), ((), ())),
                             preferred_element_type=_f32)
    pre = g1[...] + g2[...] + eaterm + dist * wd[...] + be1[...]
    h = pre * jax.nn.sigmoid(pre)
    hb = h.astype(jnp.bfloat16)
    ef = jnp.dot(hb, we2[...].astype(jnp.bfloat16),
                 preferred_element_type=_f32) + be2[...]
    ef_o[...] = ef
    cv = jnp.dot(ef.astype(jnp.bfloat16), wc1[...].astype(jnp.bfloat16),
                 preferred_element_type=_f32) + bc1[...]
    cs = cv * jax.nn.sigmoid(cv)
    sc = jnp.dot(cs, wc2[...], preferred_element_type=_f32)
    cu = diff * (sc / (jnp.sqrt(dist) + 1e-8))
    cu_o[...] = jnp.concatenate(
        [cu, jnp.zeros((cu.shape[0], D - ED), _f32)], axis=1)


def _edge_mlp(g1, g2, crc, eat, we1e, wd, be1, we2, be2, wc1, bc1, wc2):
    bs = 512
    nb = E // bs
    full = lambda r, c: pl.BlockSpec((r, c), lambda i: (0, 0))
    return pl.pallas_call(
        _edge_body,
        grid=(nb,),
        in_specs=[
            pl.BlockSpec((bs, D), lambda i: (i, 0)),
            pl.BlockSpec((bs, D), lambda i: (i, 0)),
            pl.BlockSpec((bs, D), lambda i: (i, 0)),
            pl.BlockSpec((ED, bs), lambda i: (0, i)),
            full(ED, D), full(1, D), full(1, D), full(D, D), full(1, D),
            full(D, D), full(1, D), full(D, 1),
        ],
        out_specs=[
            pl.BlockSpec((bs, D), lambda i: (i, 0)),
            pl.BlockSpec((bs, D), lambda i: (i, 0)),
        ],
        out_shape=[
            jax.ShapeDtypeStruct((E, D), _f32),
            jax.ShapeDtypeStruct((E, D), _f32),
        ],
    )(g1, g2, crc, eat, we1e, wd, be1, we2, be2, wc1, bc1, wc2)


def _make_scatter_body(width, lanes):
    def body(efh, rowg, zrows, aggo, idx, ef, acc, rs0, rs1):
        c = lax.axis_index("c")
        s = lax.axis_index("s")
        wid = s * NC + c
        pltpu.sync_copy(zrows, acc.at[pl.ds(s * RPT, RPT)])
        pltpu.sync_copy(rowg.at[pl.ds(wid * BS, BS)], idx.at[pl.ds(0, BS)])
        plsc.subcore_barrier()
        rsem = (rs0, rs1)

        def src(j):
            base = pl.multiple_of((wid * BS + j) * TB, TB)
            if lanes == width:
                return efh.at[pl.ds(base, TB)]
            return efh.at[pl.ds(base, TB), pl.ds(0, lanes)]

        def src0():
            if lanes == width:
                return efh.at[pl.ds(0, TB)]
            return efh.at[pl.ds(0, TB), pl.ds(0, lanes)]

        def rfire(j, b):
            pltpu.async_copy(src(j), ef.at[b], rsem[b])

        def rdrain(b):
            pltpu.make_async_copy(src0(), ef.at[b], rsem[b]).wait()

        rfire(0, 0)
        rfire(1, 1)

        @pl.loop(0, BS - 3, step=2)
        def _batch(j):
            for b in range(2):
                jj = j + b
                rdrain(b)
                pltpu.sync_copy(ef.at[b], acc.at[idx.at[jj]], add=True)
                rfire(jj + 2, b)

        for b in range(2):
            jj = BS - 3 + b
            rdrain(b)
            pltpu.sync_copy(ef.at[b], acc.at[idx.at[jj]], add=True)
            if b == 0:
                rfire(BS - 1, 0)
        rdrain(0)
        pltpu.sync_copy(ef.at[0], acc.at[idx.at[BS - 1]], add=True)

        plsc.subcore_barrier()
        pltpu.sync_copy(acc.at[pl.ds(s * RPT, RPT)],
                        aggo.at[c].at[pl.ds(s * RPT, RPT)])

    return body


def _scatter(efh, rowg, zrows, width, lanes):
    mesh = plsc.VectorSubcoreMesh(core_axis_name="c", subcore_axis_name="s")
    fn = pl.kernel(
        _make_scatter_body(width, lanes),
        out_type=jax.ShapeDtypeStruct((NC, NP, lanes), _f32),
        mesh=mesh,
        scratch_types=[
            pltpu.VMEM((BS, TB), _i32),
            pltpu.VMEM((2, TB, lanes), _f32),
            pltpu.VMEM_SHARED((NP, lanes), _f32),
            pltpu.SemaphoreType.DMA,
            pltpu.SemaphoreType.DMA,
        ],
        compiler_params=_SC_PARAMS,
    )
    return fn(efh, rowg, zrows)


def _node_body(x, a0, a1, cg0, cg1, c16, wn1x, wn1a, bn1, wn2, bn2,
               xn_o, cn_o):
    agg = a0[...] + a1[...]
    t = (jnp.dot(x[...], wn1x[...], preferred_element_type=_f32)
         + jnp.dot(agg, wn1a[...], preferred_element_type=_f32) + bn1[...])
    nmid = t * jax.nn.sigmoid(t)
    xn_o[...] = jnp.dot(nmid, wn2[...], preferred_element_type=_f32) + bn2[...]
    cn_o[...] = c16[...] + cg0[...] + cg1[...]


def _node_mlp(x, a0, a1, cg0, cg1, c16, wn1x, wn1a, bn1, wn2, bn2):
    nb = 10
    bs = N // nb
    full = lambda r, c: pl.BlockSpec((r, c), lambda i: (0, 0))
    return pl.pallas_call(
        _node_body,
        grid=(nb,),
        in_specs=[
            pl.BlockSpec((bs, D), lambda i: (i, 0)),
            pl.BlockSpec((bs, D), lambda i: (i, 0)),
            pl.BlockSpec((bs, D), lambda i: (i, 0)),
            pl.BlockSpec((bs, ED), lambda i: (i, 0)),
            pl.BlockSpec((bs, ED), lambda i: (i, 0)),
            pl.BlockSpec((bs, ED), lambda i: (i, 0)),
            full(D, D), full(D, D), full(1, D), full(D, D), full(1, D),
        ],
        out_specs=[
            pl.BlockSpec((bs, D), lambda i: (i, 0)),
            pl.BlockSpec((bs, ED), lambda i: (i, 0)),
        ],
        out_shape=[
            jax.ShapeDtypeStruct((N, D), _f32),
            jax.ShapeDtypeStruct((N, ED), _f32),
        ],
    )(x, a0, a1, cg0, cg1, c16, wn1x, wn1a, bn1, wn2, bn2)


def kernel(x, edge_index, coords, edge_attr,
           We1, be1, We2, be2, Wc1, bc1, Wc2, Wn1, bn1, Wn2, bn2):
    row = edge_index[0].astype(_i32)
    col = edge_index[1].astype(_i32)
    rowg = row.reshape(NBT, TB)
    colg = col.reshape(NBT, TB)
    c16 = jnp.pad(coords, ((0, 0), (0, ED - 3)))

    xr1, xc1 = _node_pre(x, We1[:D], We1[D:2 * D])
    g1, g2, crc = _edge_gather(xr1, xc1, c16, rowg, colg)

    we1e = We1[2 * D:2 * D + ED]
    wd = We1[2 * D + ED:]
    ef, cu = _edge_mlp(g1, g2, crc, edge_attr.T,
                       we1e, wd, be1.reshape(1, D), We2, be2.reshape(1, D),
                       Wc1, bc1.reshape(1, D), Wc2)

    z128 = jnp.zeros((RPT, D), _f32)
    z16 = jnp.zeros((RPT, ED), _f32)
    agg = _scatter(ef, rowg, z128, D, D)
    cag = _scatter(cu, rowg, z16, D, ED)

    xn, cn = _node_mlp(x, agg[0, :N], agg[1, :N], cag[0, :N], cag[1, :N],
                       c16, Wn1[:D], Wn1[D:], bn1.reshape(1, D),
                       Wn2, bn2.reshape(1, D))
    return (xn, cn[:, :3])


# speedup vs baseline: 4.1555x; 1.0689x over previous
import jax
import jax.numpy as jnp
from jax import lax
from jax.experimental import pallas as pl
from jax.experimental.pallas import tpu as pltpu
from jax.experimental.pallas import tpu_sc as plsc

N = 10000
E = 320000
D = 128
ED = 16

NC = 2
NS = 16
TB = 80
NBT = E // TB
NP = 10240
RPT = NP // NS

CH = (
    (0, 86, 42),
    (2048, 82, 40),
)
_BS = 512

_f32 = jnp.float32
_i32 = jnp.int32

_SC_PARAMS = pltpu.CompilerParams(use_tc_tiling_on_sc=False)


def _pre_body(x_ref, wr_ref, wc_ref, xr_ref, xc_ref):
    xb = x_ref[...]
    xr_ref[...] = jnp.dot(xb, wr_ref[...], preferred_element_type=_f32)
    xc_ref[...] = jnp.dot(xb, wc_ref[...], preferred_element_type=_f32)


def _node_pre(x, We1_r, We1_c):
    nb = 10
    bs = N // nb
    return pl.pallas_call(
        _pre_body,
        grid=(nb,),
        in_specs=[
            pl.BlockSpec((bs, D), lambda i: (i, 0)),
            pl.BlockSpec((D, D), lambda i: (0, 0)),
            pl.BlockSpec((D, D), lambda i: (0, 0)),
        ],
        out_specs=[
            pl.BlockSpec((bs, D), lambda i: (i, 0)),
            pl.BlockSpec((bs, D), lambda i: (i, 0)),
        ],
        out_shape=[
            jax.ShapeDtypeStruct((N, D), _f32),
            jax.ShapeDtypeStruct((N, D), _f32),
        ],
    )(x, We1_r, We1_c)


def _make_gather_body(boff, b0, b1):
    nb0t = NS * b0

    def body(xr1, xc1, c16, rowg, colg, g1o, g2o, crco,
             idxr, idxc, g1, g2, cr, cc, gs0, gs1, ws0, ws1):
        c = lax.axis_index("c")
        s = lax.axis_index("s")
        gsem = (gs0, gs1)
        wsem = (ws0, ws1)
        nb = jnp.where(c == 0, b0, b1)
        off = boff + jnp.where(c == 0, s * b0, nb0t + s * b1)
        roff = off - boff
        pltpu.sync_copy(rowg.at[pl.ds(off, b1)], idxr.at[pl.ds(0, b1)])
        pltpu.sync_copy(colg.at[pl.ds(off, b1)], idxc.at[pl.ds(0, b1)])

        @pl.when(c == 0)
        def _rest():
            pltpu.sync_copy(rowg.at[pl.ds(off + b1, b0 - b1)],
                            idxr.at[pl.ds(b1, b0 - b1)])
            pltpu.sync_copy(colg.at[pl.ds(off + b1, b0 - b1)],
                            idxc.at[pl.ds(b1, b0 - b1)])

        def gfire(j, b):
            pltpu.async_copy(xr1.at[idxr.at[j]], g1.at[b], gsem[b])
            pltpu.async_copy(xc1.at[idxc.at[j]], g2.at[b], gsem[b])
            pltpu.async_copy(c16.at[idxr.at[j]], cr.at[b], gsem[b])
            pltpu.async_copy(c16.at[idxc.at[j]], cc.at[b], gsem[b])

        def gdrain(b):
            pltpu.make_async_copy(xr1.at[pl.ds(0, TB)], g1.at[b],
                                  gsem[b]).wait()
            pltpu.make_async_copy(xc1.at[pl.ds(0, TB)], g2.at[b],
                                  gsem[b]).wait()
            pltpu.make_async_copy(c16.at[pl.ds(0, TB)], cr.at[b],
                                  gsem[b]).wait()
            pltpu.make_async_copy(c16.at[pl.ds(0, TB)], cc.at[b],
                                  gsem[b]).wait()

        def wfire(j, b):
            base = pl.multiple_of((roff + j) * TB, TB)
            pltpu.async_copy(g1.at[b], g1o.at[pl.ds(base, TB)], wsem[b])
            pltpu.async_copy(g2.at[b], g2o.at[pl.ds(base, TB)], wsem[b])
            pltpu.async_copy(cr.at[b],
                             crco.at[pl.ds(base, TB), pl.ds(0, ED)], wsem[b])
            pltpu.async_copy(cc.at[b],
                             crco.at[pl.ds(base, TB), pl.ds(ED, ED)], wsem[b])

        def wdrain(b):
            pltpu.make_async_copy(g1.at[b], g1o.at[pl.ds(0, TB)],
                                  wsem[b]).wait()
            pltpu.make_async_copy(g2.at[b], g2o.at[pl.ds(0, TB)],
                                  wsem[b]).wait()
            pltpu.make_async_copy(cr.at[b],
                                  crco.at[pl.ds(0, TB), pl.ds(0, ED)],
                                  wsem[b]).wait()
            pltpu.make_async_copy(cc.at[b],
                                  crco.at[pl.ds(0, TB), pl.ds(ED, ED)],
                                  wsem[b]).wait()

        gfire(0, 0)
        gfire(1, 1)

        @pl.loop(0, nb - 2, step=2)
        def _batch(j):
            for b in range(2):
                jj = j + b
                gdrain(b)
                wfire(jj, b)
                wdrain(b)
                gfire(jj + 2, b)

        for b in range(2):
            gdrain(b)
            wfire(nb - 2 + b, b)
            wdrain(b)

    return body


def _edge_gather(xr1, xc1, c16, rowg, colg, boff, b0, b1):
    ne = NS * (b0 + b1) * TB
    mesh = plsc.VectorSubcoreMesh(core_axis_name="c", subcore_axis_name="s")
    fn = pl.kernel(
        _make_gather_body(boff, b0, b1),
        out_type=[
            jax.ShapeDtypeStruct((ne, D), _f32),
            jax.ShapeDtypeStruct((ne, D), _f32),
            jax.ShapeDtypeStruct((ne, D), _f32),
        ],
        mesh=mesh,
        scratch_types=[
            pltpu.VMEM((b0, TB), _i32),
            pltpu.VMEM((b0, TB), _i32),
            pltpu.VMEM((2, TB, D), _f32),
            pltpu.VMEM((2, TB, D), _f32),
            pltpu.VMEM((2, TB, ED), _f32),
            pltpu.VMEM((2, TB, ED), _f32),
            pltpu.SemaphoreType.DMA,
            pltpu.SemaphoreType.DMA,
            pltpu.SemaphoreType.DMA,
            pltpu.SemaphoreType.DMA,
        ],
        compiler_params=_SC_PARAMS,
    )
    return fn(xr1, xc1, c16, rowg, colg)


def _edge_body(g1, g2, crc, eat, we1e, wd, be1, we2, be2, wc1, bc1, wc2,
               ef_o, cu_o):
    crcv = crc[...]
    diff = crcv[:, 0:ED] - crcv[:, ED:2 * ED]
    dist = jnp.sum(diff * diff, axis=1, keepdims=True)
    eaterm = lax.dot_general(eat[...], we1e[...], (((0,), (0,)), ((), ())),
                             preferred_element_type=_f32)
    pre = g1[...] + g2[...] + eaterm + dist * wd[...] + be1[...]
    h = pre * jax.nn.sigmoid(pre)
    hb = h.astype(jnp.bfloat16)
    ef = jnp.dot(hb, we2[...].astype(jnp.bfloat16),
                 preferred_element_type=_f32) + be2[...]
    ef_o[...] = ef
    cv = jnp.dot(ef.astype(jnp.bfloat16), wc1[...].astype(jnp.bfloat16),
                 preferred_element_type=_f32) + bc1[...]
    cs = cv * jax.nn.sigmoid(cv)
    sc = jnp.dot(cs, wc2[...], preferred_element_type=_f32)
    cu = diff * (sc / (jnp.sqrt(dist) + 1e-8))
    cu_o[...] = jnp.concatenate(
        [cu, jnp.zeros((cu.shape[0], D - ED), _f32)], axis=1)


def _edge_mlp(g1, g2, crc, eat, we1e, wd, be1, we2, be2, wc1, bc1, wc2,
              eoff):
    ne = g1.shape[0]
    nb = ne // _BS
    ob = eoff // _BS
    full = lambda r, c: pl.BlockSpec((r, c), lambda i: (0, 0))
    return pl.pallas_call(
        _edge_body,
        grid=(nb,),
        in_specs=[
            pl.BlockSpec((_BS, D), lambda i: (i, 0)),
            pl.BlockSpec((_BS, D), lambda i: (i, 0)),
            pl.BlockSpec((_BS, D), lambda i: (i, 0)),
            pl.BlockSpec((ED, _BS), lambda i: (0, i + ob)),
            full(ED, D), full(1, D), full(1, D), full(D, D), full(1, D),
            full(D, D), full(1, D), full(D, 1),
        ],
        out_specs=[
            pl.BlockSpec((_BS, D), lambda i: (i, 0)),
            pl.BlockSpec((_BS, D), lambda i: (i, 0)),
        ],
        out_shape=[
            jax.ShapeDtypeStruct((ne, D), _f32),
            jax.ShapeDtypeStruct((ne, D), _f32),
        ],
    )(g1, g2, crc, eat, we1e, wd, be1, we2, be2, wc1, bc1, wc2)


def _make_scatter_body(width, lanes, boff, bs_c):
    def body(efh, rowg, zrows, aggo, idx, ef, acc, rs0, rs1):
        c = lax.axis_index("c")
        s = lax.axis_index("s")
        wid = s * NC + c
        pltpu.sync_copy(zrows, acc.at[pl.ds(s * RPT, RPT)])
        pltpu.sync_copy(rowg.at[pl.ds(boff + wid * bs_c, bs_c)], idx)
        plsc.subcore_barrier()
        rsem = (rs0, rs1)

        def src(j):
            base = pl.multiple_of((wid * bs_c + j) * TB, TB)
            if lanes == width:
                return efh.at[pl.ds(base, TB)]
            return efh.at[pl.ds(base, TB), pl.ds(0, lanes)]

        def src0():
            if lanes == width:
                return efh.at[pl.ds(0, TB)]
            return efh.at[pl.ds(0, TB), pl.ds(0, lanes)]

        def rfire(j, b):
            pltpu.async_copy(src(j), ef.at[b], rsem[b])

        def rdrain(b):
            pltpu.make_async_copy(src0(), ef.at[b], rsem[b]).wait()

        def scat(j, b):
            pltpu.sync_copy(ef.at[b], acc.at[idx.at[j]], add=True)

        rfire(0, 0)
        rfire(1, 1)

        if bs_c % 2 == 0:
            @pl.loop(0, bs_c - 2, step=2)
            def _batch(j):
                for b in range(2):
                    jj = j + b
                    rdrain(b)
                    scat(jj, b)
                    rfire(jj + 2, b)

            for b in range(2):
                rdrain(b)
                scat(bs_c - 2 + b, b)
        else:
            @pl.loop(0, bs_c - 3, step=2)
            def _batch(j):
                for b in range(2):
                    jj = j + b
                    rdrain(b)
                    scat(jj, b)
                    rfire(jj + 2, b)

            for b in range(2):
                rdrain(b)
                scat(bs_c - 3 + b, b)
                if b == 0:
                    rfire(bs_c - 1, 0)
            rdrain(0)
            scat(bs_c - 1, 0)

        plsc.subcore_barrier()
        pltpu.sync_copy(acc.at[pl.ds(s * RPT, RPT)],
                        aggo.at[c].at[pl.ds(s * RPT, RPT)])

    return body


def _scatter(efh, rowg, zrows, width, lanes, boff, nbatch):
    bs_c = nbatch // (NC * NS)
    mesh = plsc.VectorSubcoreMesh(core_axis_name="c", subcore_axis_name="s")
    fn = pl.kernel(
        _make_scatter_body(width, lanes, boff, bs_c),
        out_type=jax.ShapeDtypeStruct((NC, NP, lanes), _f32),
        mesh=mesh,
        scratch_types=[
            pltpu.VMEM((bs_c, TB), _i32),
            pltpu.VMEM((2, TB, lanes), _f32),
            pltpu.VMEM_SHARED((NP, lanes), _f32),
            pltpu.SemaphoreType.DMA,
            pltpu.SemaphoreType.DMA,
        ],
        compiler_params=_SC_PARAMS,
    )
    return fn(efh, rowg, zrows)


def _node_body(x, a0, a1, a2, a3, cg0, cg1, cg2, cg3, c16,
               wn1x, wn1a, bn1, wn2, bn2, xn_o, cn_o):
    agg = (a0[...] + a1[...]) + (a2[...] + a3[...])
    t = (jnp.dot(x[...], wn1x[...], preferred_element_type=_f32)
         + jnp.dot(agg, wn1a[...], preferred_element_type=_f32) + bn1[...])
    nmid = t * jax.nn.sigmoid(t)
    xn_o[...] = jnp.dot(nmid, wn2[...], preferred_element_type=_f32) + bn2[...]
    cn_o[...] = (c16[...] + cg0[...] + cg1[...]) + (cg2[...] + cg3[...])


def _node_mlp(x, aggs, cags, c16, wn1x, wn1a, bn1, wn2, bn2):
    nb = 10
    bs = N // nb
    full = lambda r, c: pl.BlockSpec((r, c), lambda i: (0, 0))
    return pl.pallas_call(
        _node_body,
        grid=(nb,),
        in_specs=[
            pl.BlockSpec((bs, D), lambda i: (i, 0)),
            pl.BlockSpec((bs, D), lambda i: (i, 0)),
            pl.BlockSpec((bs, D), lambda i: (i, 0)),
            pl.BlockSpec((bs, D), lambda i: (i, 0)),
            pl.BlockSpec((bs, D), lambda i: (i, 0)),
            pl.BlockSpec((bs, ED), lambda i: (i, 0)),
            pl.BlockSpec((bs, ED), lambda i: (i, 0)),
            pl.BlockSpec((bs, ED), lambda i: (i, 0)),
            pl.BlockSpec((bs, ED), lambda i: (i, 0)),
            pl.BlockSpec((bs, ED), lambda i: (i, 0)),
            full(D, D), full(D, D), full(1, D), full(D, D), full(1, D),
        ],
        out_specs=[
            pl.BlockSpec((bs, D), lambda i: (i, 0)),
            pl.BlockSpec((bs, ED), lambda i: (i, 0)),
        ],
        out_shape=[
            jax.ShapeDtypeStruct((N, D), _f32),
            jax.ShapeDtypeStruct((N, ED), _f32),
        ],
    )(x, *aggs, *cags, c16, wn1x, wn1a, bn1, wn2, bn2)


def kernel(x, edge_index, coords, edge_attr,
           We1, be1, We2, be2, Wc1, bc1, Wc2, Wn1, bn1, Wn2, bn2):
    row = edge_index[0].astype(_i32)
    col = edge_index[1].astype(_i32)
    rowg = row.reshape(NBT, TB)
    colg = col.reshape(NBT, TB)
    c16 = jnp.pad(coords, ((0, 0), (0, ED - 3)))
    eat = edge_attr.T

    xr1, xc1 = _node_pre(x, We1[:D], We1[D:2 * D])

    we1e = We1[2 * D:2 * D + ED]
    wd = We1[2 * D + ED:]
    z128 = jnp.zeros((RPT, D), _f32)
    z16 = jnp.zeros((RPT, ED), _f32)

    aggs, cags = [], []
    for boff, b0, b1 in CH:
        nbatch = NS * (b0 + b1)
        g1, g2, crc = _edge_gather(xr1, xc1, c16, rowg, colg, boff, b0, b1)
        ef, cu = _edge_mlp(g1, g2, crc, eat, we1e, wd, be1.reshape(1, D),
                           We2, be2.reshape(1, D), Wc1, bc1.reshape(1, D),
                           Wc2, boff * TB)
        aggs.append(_scatter(ef, rowg, z128, D, D, boff, nbatch))
        cags.append(_scatter(cu, rowg, z16, D, ED, boff, nbatch))

    a = [p[i, :N] for p in aggs for i in range(NC)]
    cg = [p[i, :N] for p in cags for i in range(NC)]
    xn, cn = _node_mlp(x, a, cg, c16, Wn1[:D], Wn1[D:], bn1.reshape(1, D),
                       Wn2, bn2.reshape(1, D))
    return (xn, cn[:, :3])


# speedup vs baseline: 4.4598x; 1.0732x over previous
import jax
import jax.numpy as jnp
from jax import lax
from jax.experimental import pallas as pl
from jax.experimental.pallas import tpu as pltpu
from jax.experimental.pallas import tpu_sc as plsc

N = 10000
E = 320000
D = 128
ED = 16

NC = 2
NS = 16
TB = 80
NBT = E // TB
NP = 10240
RPT = NP // NS

CH = (
    (0, 70, 58),
    (2048, 68, 54),
)
_BS = 640

_f32 = jnp.float32
_i32 = jnp.int32

_SC_PARAMS = pltpu.CompilerParams(use_tc_tiling_on_sc=False)


def _pre_body(x_ref, wr_ref, wc_ref, xr_ref, xc_ref):
    xb = x_ref[...]
    xr_ref[...] = jnp.dot(xb, wr_ref[...], preferred_element_type=_f32)
    xc_ref[...] = jnp.dot(xb, wc_ref[...], preferred_element_type=_f32)


def _node_pre(x, We1_r, We1_c):
    nb = 10
    bs = N // nb
    return pl.pallas_call(
        _pre_body,
        grid=(nb,),
        in_specs=[
            pl.BlockSpec((bs, D), lambda i: (i, 0)),
            pl.BlockSpec((D, D), lambda i: (0, 0)),
            pl.BlockSpec((D, D), lambda i: (0, 0)),
        ],
        out_specs=[
            pl.BlockSpec((bs, D), lambda i: (i, 0)),
            pl.BlockSpec((bs, D), lambda i: (i, 0)),
        ],
        out_shape=[
            jax.ShapeDtypeStruct((N, D), _f32),
            jax.ShapeDtypeStruct((N, D), _f32),
        ],
    )(x, We1_r, We1_c)


def _make_gather_body(boff, b0, b1):
    nb0t = NS * b0

    def body(xr1, xc1, c16, rowg, colg, g1o, g2o, crco,
             idxr, idxc, g1, g2, cr, cc, gs0, gs1, ws0, ws1):
        c = lax.axis_index("c")
        s = lax.axis_index("s")
        gsem = (gs0, gs1)
        wsem = (ws0, ws1)
        nb = jnp.where(c == 0, b0, b1)
        off = boff + jnp.where(c == 0, s * b0, nb0t + s * b1)
        roff = off - boff
        pltpu.sync_copy(rowg.at[pl.ds(off, b1)], idxr.at[pl.ds(0, b1)])
        pltpu.sync_copy(colg.at[pl.ds(off, b1)], idxc.at[pl.ds(0, b1)])

        @pl.when(c == 0)
        def _rest():
            pltpu.sync_copy(rowg.at[pl.ds(off + b1, b0 - b1)],
                            idxr.at[pl.ds(b1, b0 - b1)])
            pltpu.sync_copy(colg.at[pl.ds(off + b1, b0 - b1)],
                            idxc.at[pl.ds(b1, b0 - b1)])

        def gfire(j, b):
            pltpu.async_copy(xr1.at[idxr.at[j]], g1.at[b], gsem[b])
            pltpu.async_copy(xc1.at[idxc.at[j]], g2.at[b], gsem[b])
            pltpu.async_copy(c16.at[idxr.at[j]], cr.at[b], gsem[b])
            pltpu.async_copy(c16.at[idxc.at[j]], cc.at[b], gsem[b])

        def gdrain(b):
            pltpu.make_async_copy(xr1.at[pl.ds(0, TB)], g1.at[b],
                                  gsem[b]).wait()
            pltpu.make_async_copy(xc1.at[pl.ds(0, TB)], g2.at[b],
                                  gsem[b]).wait()
            pltpu.make_async_copy(c16.at[pl.ds(0, TB)], cr.at[b],
                                  gsem[b]).wait()
            pltpu.make_async_copy(c16.at[pl.ds(0, TB)], cc.at[b],
                                  gsem[b]).wait()

        def wfire(j, b):
            base = pl.multiple_of((roff + j) * TB, TB)
            pltpu.async_copy(g1.at[b], g1o.at[pl.ds(base, TB)], wsem[b])
            pltpu.async_copy(g2.at[b], g2o.at[pl.ds(base, TB)], wsem[b])
            pltpu.async_copy(cr.at[b],
                             crco.at[pl.ds(base, TB), pl.ds(0, ED)], wsem[b])
            pltpu.async_copy(cc.at[b],
                             crco.at[pl.ds(base, TB), pl.ds(ED, ED)], wsem[b])

        def wdrain(b):
            pltpu.make_async_copy(g1.at[b], g1o.at[pl.ds(0, TB)],
                                  wsem[b]).wait()
            pltpu.make_async_copy(g2.at[b], g2o.at[pl.ds(0, TB)],
                                  wsem[b]).wait()
            pltpu.make_async_copy(cr.at[b],
                                  crco.at[pl.ds(0, TB), pl.ds(0, ED)],
                                  wsem[b]).wait()
            pltpu.make_async_copy(cc.at[b],
                                  crco.at[pl.ds(0, TB), pl.ds(ED, ED)],
                                  wsem[b]).wait()

        gfire(0, 0)
        gfire(1, 1)

        @pl.loop(0, nb - 2, step=2)
        def _batch(j):
            for b in range(2):
                jj = j + b
                gdrain(b)
                wfire(jj, b)
                wdrain(b)
                gfire(jj + 2, b)

        for b in range(2):
            gdrain(b)
            wfire(nb - 2 + b, b)
            wdrain(b)

    return body


def _edge_gather(xr1, xc1, c16, rowg, colg, boff, b0, b1):
    ne = NS * (b0 + b1) * TB
    mesh = plsc.VectorSubcoreMesh(core_axis_name="c", subcore_axis_name="s")
    fn = pl.kernel(
        _make_gather_body(boff, b0, b1),
        out_type=[
            jax.ShapeDtypeStruct((ne, D), _f32),
            jax.ShapeDtypeStruct((ne, D), _f32),
            jax.ShapeDtypeStruct((ne, D), _f32),
        ],
        mesh=mesh,
        scratch_types=[
            pltpu.VMEM((b0, TB), _i32),
            pltpu.VMEM((b0, TB), _i32),
            pltpu.VMEM((2, TB, D), _f32),
            pltpu.VMEM((2, TB, D), _f32),
            pltpu.VMEM((2, TB, ED), _f32),
            pltpu.VMEM((2, TB, ED), _f32),
            pltpu.SemaphoreType.DMA,
            pltpu.SemaphoreType.DMA,
            pltpu.SemaphoreType.DMA,
            pltpu.SemaphoreType.DMA,
        ],
        compiler_params=_SC_PARAMS,
    )
    return fn(xr1, xc1, c16, rowg, colg)


def _edge_body(g1, g2, crc, eat, we1e, wd, be1, we2, be2, wc1, bc1, wc2,
               ef_o, cu_o):
    crcv = crc[...]
    diff = crcv[:, 0:ED] - crcv[:, ED:2 * ED]
    dist = jnp.sum(diff * diff, axis=1, keepdims=True)
    eaterm = lax.dot_general(eat[...], we1e[...], (((0,), (0,)), ((), ())),
                             preferred_element_type=_f32)
    pre = g1[...] + g2[...] + eaterm + dist * wd[...] + be1[...]
    h = pre * jax.nn.sigmoid(pre)
    hb = h.astype(jnp.bfloat16)
    ef = jnp.dot(hb, we2[...].astype(jnp.bfloat16),
                 preferred_element_type=_f32) + be2[...]
    ef_o[...] = ef
    cv = jnp.dot(ef.astype(jnp.bfloat16), wc1[...].astype(jnp.bfloat16),
                 preferred_element_type=_f32) + bc1[...]
    cs = cv * jax.nn.sigmoid(cv)
    sc = jnp.dot(cs, wc2[...], preferred_element_type=_f32)
    cu = diff * (sc / (jnp.sqrt(dist) + 1e-8))
    cu_o[...] = jnp.concatenate(
        [cu, jnp.zeros((cu.shape[0], D - ED), _f32)], axis=1)


def _edge_mlp(g1, g2, crc, eat, we1e, wd, be1, we2, be2, wc1, bc1, wc2,
              eoff):
    ne = g1.shape[0]
    nb = ne // _BS
    ob = eoff // _BS
    full = lambda r, c: pl.BlockSpec((r, c), lambda i: (0, 0))
    return pl.pallas_call(
        _edge_body,
        grid=(nb,),
        in_specs=[
            pl.BlockSpec((_BS, D), lambda i: (i, 0)),
            pl.BlockSpec((_BS, D), lambda i: (i, 0)),
            pl.BlockSpec((_BS, D), lambda i: (i, 0)),
            pl.BlockSpec((ED, _BS), lambda i: (0, i + ob)),
            full(ED, D), full(1, D), full(1, D), full(D, D), full(1, D),
            full(D, D), full(1, D), full(D, 1),
        ],
        out_specs=[
            pl.BlockSpec((_BS, D), lambda i: (i, 0)),
            pl.BlockSpec((_BS, D), lambda i: (i, 0)),
        ],
        out_shape=[
            jax.ShapeDtypeStruct((ne, D), _f32),
            jax.ShapeDtypeStruct((ne, D), _f32),
        ],
    )(g1, g2, crc, eat, we1e, wd, be1, we2, be2, wc1, bc1, wc2)


def _make_scatter_body(width, lanes, boff, bs_c):
    def body(efh, rowg, zrows, aggo, idx, ef, acc, rs0, rs1):
        c = lax.axis_index("c")
        s = lax.axis_index("s")
        wid = s * NC + c
        pltpu.sync_copy(zrows, acc.at[pl.ds(s * RPT, RPT)])
        pltpu.sync_copy(rowg.at[pl.ds(boff + wid * bs_c, bs_c)], idx)
        plsc.subcore_barrier()
        rsem = (rs0, rs1)

        def src(j):
            base = pl.multiple_of((wid * bs_c + j) * TB, TB)
            if lanes == width:
                return efh.at[pl.ds(base, TB)]
            return efh.at[pl.ds(base, TB), pl.ds(0, lanes)]

        def src0():
            if lanes == width:
                return efh.at[pl.ds(0, TB)]
            return efh.at[pl.ds(0, TB), pl.ds(0, lanes)]

        def rfire(j, b):
            pltpu.async_copy(src(j), ef.at[b], rsem[b])

        def rdrain(b):
            pltpu.make_async_copy(src0(), ef.at[b], rsem[b]).wait()

        def scat(j, b):
            pltpu.sync_copy(ef.at[b], acc.at[idx.at[j]], add=True)

        rfire(0, 0)
        rfire(1, 1)

        if bs_c % 2 == 0:
            @pl.loop(0, bs_c - 2, step=2)
            def _batch(j):
                for b in range(2):
                    jj = j + b
                    rdrain(b)
                    scat(jj, b)
                    rfire(jj + 2, b)

            for b in range(2):
                rdrain(b)
                scat(bs_c - 2 + b, b)
        else:
            @pl.loop(0, bs_c - 3, step=2)
            def _batch(j):
                for b in range(2):
                    jj = j + b
                    rdrain(b)
                    scat(jj, b)
                    rfire(jj + 2, b)

            for b in range(2):
                rdrain(b)
                scat(bs_c - 3 + b, b)
                if b == 0:
                    rfire(bs_c - 1, 0)
            rdrain(0)
            scat(bs_c - 1, 0)

        plsc.subcore_barrier()
        pltpu.sync_copy(acc.at[pl.ds(s * RPT, RPT)],
                        aggo.at[c].at[pl.ds(s * RPT, RPT)])

    return body


def _scatter(efh, rowg, zrows, width, lanes, boff, nbatch):
    bs_c = nbatch // (NC * NS)
    mesh = plsc.VectorSubcoreMesh(core_axis_name="c", subcore_axis_name="s")
    fn = pl.kernel(
        _make_scatter_body(width, lanes, boff, bs_c),
        out_type=jax.ShapeDtypeStruct((NC, NP, lanes), _f32),
        mesh=mesh,
        scratch_types=[
            pltpu.VMEM((bs_c, TB), _i32),
            pltpu.VMEM((2, TB, lanes), _f32),
            pltpu.VMEM_SHARED((NP, lanes), _f32),
            pltpu.SemaphoreType.DMA,
            pltpu.SemaphoreType.DMA,
        ],
        compiler_params=_SC_PARAMS,
    )
    return fn(efh, rowg, zrows)


def _node_body(x, a0, a1, a2, a3, cg0, cg1, cg2, cg3, c16,
               wn1x, wn1a, bn1, wn2, bn2, xn_o, cn_o):
    agg = (a0[...] + a1[...]) + (a2[...] + a3[...])
    t = (jnp.dot(x[...], wn1x[...], preferred_element_type=_f32)
         + jnp.dot(agg, wn1a[...], preferred_element_type=_f32) + bn1[...])
    nmid = t * jax.nn.sigmoid(t)
    xn_o[...] = jnp.dot(nmid, wn2[...], preferred_element_type=_f32) + bn2[...]
    cn_o[...] = (c16[...] + cg0[...] + cg1[...]) + (cg2[...] + cg3[...])


def _node_mlp(x, aggs, cags, c16, wn1x, wn1a, bn1, wn2, bn2):
    nb = 10
    bs = N // nb
    full = lambda r, c: pl.BlockSpec((r, c), lambda i: (0, 0))
    return pl.pallas_call(
        _node_body,
        grid=(nb,),
        in_specs=[
            pl.BlockSpec((bs, D), lambda i: (i, 0)),
            pl.BlockSpec((bs, D), lambda i: (i, 0)),
            pl.BlockSpec((bs, D), lambda i: (i, 0)),
            pl.BlockSpec((bs, D), lambda i: (i, 0)),
            pl.BlockSpec((bs, D), lambda i: (i, 0)),
            pl.BlockSpec((bs, ED), lambda i: (i, 0)),
            pl.BlockSpec((bs, ED), lambda i: (i, 0)),
            pl.BlockSpec((bs, ED), lambda i: (i, 0)),
            pl.BlockSpec((bs, ED), lambda i: (i, 0)),
            pl.BlockSpec((bs, ED), lambda i: (i, 0)),
            full(D, D), full(D, D), full(1, D), full(D, D), full(1, D),
        ],
        out_specs=[
            pl.BlockSpec((bs, D), lambda i: (i, 0)),
            pl.BlockSpec((bs, ED), lambda i: (i, 0)),
        ],
        out_shape=[
            jax.ShapeDtypeStruct((N, D), _f32),
            jax.ShapeDtypeStruct((N, ED), _f32),
        ],
    )(x, *aggs, *cags, c16, wn1x, wn1a, bn1, wn2, bn2)


def kernel(x, edge_index, coords, edge_attr,
           We1, be1, We2, be2, Wc1, bc1, Wc2, Wn1, bn1, Wn2, bn2):
    row = edge_index[0].astype(_i32)
    col = edge_index[1].astype(_i32)
    rowg = row.reshape(NBT, TB)
    colg = col.reshape(NBT, TB)
    c16 = jnp.pad(coords, ((0, 0), (0, ED - 3)))
    eat = edge_attr.T

    xr1, xc1 = _node_pre(x, We1[:D], We1[D:2 * D])

    we1e = We1[2 * D:2 * D + ED]
    wd = We1[2 * D + ED:]
    z128 = jnp.zeros((RPT, D), _f32)
    z16 = jnp.zeros((RPT, ED), _f32)

    aggs, cags = [], []
    for boff, b0, b1 in CH:
        nbatch = NS * (b0 + b1)
        g1, g2, crc = _edge_gather(xr1, xc1, c16, rowg, colg, boff, b0, b1)
        ef, cu = _edge_mlp(g1, g2, crc, eat, we1e, wd, be1.reshape(1, D),
                           We2, be2.reshape(1, D), Wc1, bc1.reshape(1, D),
                           Wc2, boff * TB)
        aggs.append(_scatter(ef, rowg, z128, D, D, boff, nbatch))
        cags.append(_scatter(cu, rowg, z16, D, ED, boff, nbatch))

    a = [p[i, :N] for p in aggs for i in range(NC)]
    cg = [p[i, :N] for p in cags for i in range(NC)]
    xn, cn = _node_mlp(x, a, cg, c16, Wn1[:D], Wn1[D:], bn1.reshape(1, D),
                       Wn2, bn2.reshape(1, D))
    return (xn, cn[:, :3])


# speedup vs baseline: 5.4594x; 1.2242x over previous
import jax
import jax.numpy as jnp
from jax import lax
from jax.experimental import pallas as pl
from jax.experimental.pallas import tpu as pltpu
from jax.experimental.pallas import tpu_sc as plsc

N = 10000
E = 320000
D = 128
ED = 16

NC = 2
NS = 16
TB = 80
NBT = E // TB
NP = 10240
RPT = NP // NS

CH = (
    (0, 70, 58),
    (2048, 68, 54),
)
_BS = 1280

_f32 = jnp.float32
_i32 = jnp.int32

_SC_PARAMS = pltpu.CompilerParams(use_tc_tiling_on_sc=False)


def _pre_body(x_ref, wr_ref, wc_ref, xr_ref, xc_ref):
    xb = x_ref[...]
    xr_ref[...] = jnp.dot(xb, wr_ref[...], preferred_element_type=_f32)
    xc_ref[...] = jnp.dot(xb, wc_ref[...], preferred_element_type=_f32)


def _node_pre(x, We1_r, We1_c):
    nb = 10
    bs = N // nb
    return pl.pallas_call(
        _pre_body,
        grid=(nb,),
        in_specs=[
            pl.BlockSpec((bs, D), lambda i: (i, 0)),
            pl.BlockSpec((D, D), lambda i: (0, 0)),
            pl.BlockSpec((D, D), lambda i: (0, 0)),
        ],
        out_specs=[
            pl.BlockSpec((bs, D), lambda i: (i, 0)),
            pl.BlockSpec((bs, D), lambda i: (i, 0)),
        ],
        out_shape=[
            jax.ShapeDtypeStruct((N, D), _f32),
            jax.ShapeDtypeStruct((N, D), _f32),
        ],
    )(x, We1_r, We1_c)


def _make_gather_body(boff, b0, b1):
    nb0t = NS * b0

    def body(xr1, xc1, c16, rowg, colg, g1o, g2o, crco,
             idxr, idxc, g1, g2, cr, cc, gs0, gs1, ws0, ws1):
        c = lax.axis_index("c")
        s = lax.axis_index("s")
        gsem = (gs0, gs1)
        wsem = (ws0, ws1)
        nb = jnp.where(c == 0, b0, b1)
        off = boff + jnp.where(c == 0, s * b0, nb0t + s * b1)
        roff = off - boff
        pltpu.sync_copy(rowg.at[pl.ds(off, b1)], idxr.at[pl.ds(0, b1)])
        pltpu.sync_copy(colg.at[pl.ds(off, b1)], idxc.at[pl.ds(0, b1)])

        @pl.when(c == 0)
        def _rest():
            pltpu.sync_copy(rowg.at[pl.ds(off + b1, b0 - b1)],
                            idxr.at[pl.ds(b1, b0 - b1)])
            pltpu.sync_copy(colg.at[pl.ds(off + b1, b0 - b1)],
                            idxc.at[pl.ds(b1, b0 - b1)])

        def gfire(j, b):
            pltpu.async_copy(xr1.at[idxr.at[j]], g1.at[b], gsem[b])
            pltpu.async_copy(xc1.at[idxc.at[j]], g2.at[b], gsem[b])
            pltpu.async_copy(c16.at[idxr.at[j]], cr.at[b], gsem[b])
            pltpu.async_copy(c16.at[idxc.at[j]], cc.at[b], gsem[b])

        def gdrain(b):
            pltpu.make_async_copy(xr1.at[pl.ds(0, TB)], g1.at[b],
                                  gsem[b]).wait()
            pltpu.make_async_copy(xc1.at[pl.ds(0, TB)], g2.at[b],
                                  gsem[b]).wait()
            pltpu.make_async_copy(c16.at[pl.ds(0, TB)], cr.at[b],
                                  gsem[b]).wait()
            pltpu.make_async_copy(c16.at[pl.ds(0, TB)], cc.at[b],
                                  gsem[b]).wait()

        def wfire(j, b):
            base = pl.multiple_of((roff + j) * TB, TB)
            pltpu.async_copy(g1.at[b], g1o.at[pl.ds(base, TB)], wsem[b])
            pltpu.async_copy(g2.at[b], g2o.at[pl.ds(base, TB)], wsem[b])
            pltpu.async_copy(cr.at[b],
                             crco.at[pl.ds(base, TB), pl.ds(0, ED)], wsem[b])
            pltpu.async_copy(cc.at[b],
                             crco.at[pl.ds(base, TB), pl.ds(ED, ED)], wsem[b])

        def wdrain(b):
            pltpu.make_async_copy(g1.at[b], g1o.at[pl.ds(0, TB)],
                                  wsem[b]).wait()
            pltpu.make_async_copy(g2.at[b], g2o.at[pl.ds(0, TB)],
                                  wsem[b]).wait()
            pltpu.make_async_copy(cr.at[b],
                                  crco.at[pl.ds(0, TB), pl.ds(0, ED)],
                                  wsem[b]).wait()
            pltpu.make_async_copy(cc.at[b],
                                  crco.at[pl.ds(0, TB), pl.ds(ED, ED)],
                                  wsem[b]).wait()

        gfire(0, 0)
        gfire(1, 1)

        @pl.loop(0, nb - 2, step=2)
        def _batch(j):
            for b in range(2):
                jj = j + b
                gdrain(b)
                wfire(jj, b)
                wdrain(b)
                gfire(jj + 2, b)

        for b in range(2):
            gdrain(b)
            wfire(nb - 2 + b, b)
            wdrain(b)

    return body


def _edge_gather(xr1, xc1, c16, rowg, colg, boff, b0, b1):
    ne = NS * (b0 + b1) * TB
    mesh = plsc.VectorSubcoreMesh(core_axis_name="c", subcore_axis_name="s")
    fn = pl.kernel(
        _make_gather_body(boff, b0, b1),
        out_type=[
            jax.ShapeDtypeStruct((ne, D), _f32),
            jax.ShapeDtypeStruct((ne, D), _f32),
            jax.ShapeDtypeStruct((ne, D), _f32),
        ],
        mesh=mesh,
        scratch_types=[
            pltpu.VMEM((b0, TB), _i32),
            pltpu.VMEM((b0, TB), _i32),
            pltpu.VMEM((2, TB, D), _f32),
            pltpu.VMEM((2, TB, D), _f32),
            pltpu.VMEM((2, TB, ED), _f32),
            pltpu.VMEM((2, TB, ED), _f32),
            pltpu.SemaphoreType.DMA,
            pltpu.SemaphoreType.DMA,
            pltpu.SemaphoreType.DMA,
            pltpu.SemaphoreType.DMA,
        ],
        compiler_params=_SC_PARAMS,
    )
    return fn(xr1, xc1, c16, rowg, colg)


def _edge_body(g1, g2, crc, eat, we1e, wd, be1, we2, be2, wc1, bc1, wc2,
               ef_o, cu_o):
    crcv = crc[...]
    diff = crcv[:, 0:ED] - crcv[:, ED:2 * ED]
    dist = jnp.sum(diff * diff, axis=1, keepdims=True)
    eaterm = lax.dot_general(eat[...], we1e[...], (((0,), (0,)), ((), ())),
                             preferred_element_type=_f32)
    pre = g1[...] + g2[...] + eaterm + dist * wd[...] + be1[...]
    h = pre * jax.nn.sigmoid(pre.astype(jnp.bfloat16)).astype(_f32)
    hb = h.astype(jnp.bfloat16)
    ef = jnp.dot(hb, we2[...].astype(jnp.bfloat16),
                 preferred_element_type=_f32) + be2[...]
    ef_o[...] = ef
    cv = jnp.dot(ef.astype(jnp.bfloat16), wc1[...].astype(jnp.bfloat16),
                 preferred_element_type=_f32) + bc1[...]
    cs = cv * jax.nn.sigmoid(cv.astype(jnp.bfloat16)).astype(_f32)
    sc = jnp.dot(cs, wc2[...], preferred_element_type=_f32)
    cu = diff * (sc / (jnp.sqrt(dist) + 1e-8))
    cu_o[:, 0:ED] = cu


def _edge_mlp(g1, g2, crc, eat, we1e, wd, be1, we2, be2, wc1, bc1, wc2,
              eoff):
    ne = g1.shape[0]
    nb = ne // _BS
    ob = eoff // _BS
    full = lambda r, c: pl.BlockSpec((r, c), lambda i: (0, 0))
    return pl.pallas_call(
        _edge_body,
        grid=(nb,),
        in_specs=[
            pl.BlockSpec((_BS, D), lambda i: (i, 0)),
            pl.BlockSpec((_BS, D), lambda i: (i, 0)),
            pl.BlockSpec((_BS, D), lambda i: (i, 0)),
            pl.BlockSpec((ED, _BS), lambda i: (0, i + ob)),
            full(ED, D), full(1, D), full(1, D), full(D, D), full(1, D),
            full(D, D), full(1, D), full(D, 1),
        ],
        out_specs=[
            pl.BlockSpec((_BS, D), lambda i: (i, 0)),
            pl.BlockSpec((_BS, D), lambda i: (i, 0)),
        ],
        out_shape=[
            jax.ShapeDtypeStruct((ne, D), _f32),
            jax.ShapeDtypeStruct((ne, D), _f32),
        ],
    )(g1, g2, crc, eat, we1e, wd, be1, we2, be2, wc1, bc1, wc2)


def _make_scatter_body(width, lanes, boff, bs_c):
    def body(efh, rowg, zrows, aggo, idx, ef, acc, rs0, rs1):
        c = lax.axis_index("c")
        s = lax.axis_index("s")
        wid = s * NC + c
        pltpu.sync_copy(zrows, acc.at[pl.ds(s * RPT, RPT)])
        pltpu.sync_copy(rowg.at[pl.ds(boff + wid * bs_c, bs_c)], idx)
        plsc.subcore_barrier()
        rsem = (rs0, rs1)

        def src(j):
            base = pl.multiple_of((wid * bs_c + j) * TB, TB)
            if lanes == width:
                return efh.at[pl.ds(base, TB)]
            return efh.at[pl.ds(base, TB), pl.ds(0, lanes)]

        def src0():
            if lanes == width:
                return efh.at[pl.ds(0, TB)]
            return efh.at[pl.ds(0, TB), pl.ds(0, lanes)]

        def rfire(j, b):
            pltpu.async_copy(src(j), ef.at[b], rsem[b])

        def rdrain(b):
            pltpu.make_async_copy(src0(), ef.at[b], rsem[b]).wait()

        def scat(j, b):
            pltpu.sync_copy(ef.at[b], acc.at[idx.at[j]], add=True)

        rfire(0, 0)
        rfire(1, 1)

        if bs_c % 2 == 0:
            @pl.loop(0, bs_c - 2, step=2)
            def _batch(j):
                for b in range(2):
                    jj = j + b
                    rdrain(b)
                    scat(jj, b)
                    rfire(jj + 2, b)

            for b in range(2):
                rdrain(b)
                scat(bs_c - 2 + b, b)
        else:
            @pl.loop(0, bs_c - 3, step=2)
            def _batch(j):
                for b in range(2):
                    jj = j + b
                    rdrain(b)
                    scat(jj, b)
                    rfire(jj + 2, b)

            for b in range(2):
                rdrain(b)
                scat(bs_c - 3 + b, b)
                if b == 0:
                    rfire(bs_c - 1, 0)
            rdrain(0)
            scat(bs_c - 1, 0)

        plsc.subcore_barrier()
        pltpu.sync_copy(acc.at[pl.ds(s * RPT, RPT)],
                        aggo.at[c].at[pl.ds(s * RPT, RPT)])

    return body


def _scatter(efh, rowg, zrows, width, lanes, boff, nbatch):
    bs_c = nbatch // (NC * NS)
    mesh = plsc.VectorSubcoreMesh(core_axis_name="c", subcore_axis_name="s")
    fn = pl.kernel(
        _make_scatter_body(width, lanes, boff, bs_c),
        out_type=jax.ShapeDtypeStruct((NC, NP, lanes), _f32),
        mesh=mesh,
        scratch_types=[
            pltpu.VMEM((bs_c, TB), _i32),
            pltpu.VMEM((2, TB, lanes), _f32),
            pltpu.VMEM_SHARED((NP, lanes), _f32),
            pltpu.SemaphoreType.DMA,
            pltpu.SemaphoreType.DMA,
        ],
        compiler_params=_SC_PARAMS,
    )
    return fn(efh, rowg, zrows)


def _node_body(x, a0, a1, a2, a3, cg0, cg1, cg2, cg3, c16,
               wn1x, wn1a, bn1, wn2, bn2, xn_o, cn_o):
    agg = (a0[...] + a1[...]) + (a2[...] + a3[...])
    t = (jnp.dot(x[...], wn1x[...], preferred_element_type=_f32)
         + jnp.dot(agg, wn1a[...], preferred_element_type=_f32) + bn1[...])
    nmid = t * jax.nn.sigmoid(t)
    xn_o[...] = jnp.dot(nmid, wn2[...], preferred_element_type=_f32) + bn2[...]
    cn_o[...] = (c16[...] + cg0[...] + cg1[...]) + (cg2[...] + cg3[...])


def _node_mlp(x, aggs, cags, c16, wn1x, wn1a, bn1, wn2, bn2):
    nb = 10
    bs = N // nb
    full = lambda r, c: pl.BlockSpec((r, c), lambda i: (0, 0))
    return pl.pallas_call(
        _node_body,
        grid=(nb,),
        in_specs=[
            pl.BlockSpec((bs, D), lambda i: (i, 0)),
            pl.BlockSpec((bs, D), lambda i: (i, 0)),
            pl.BlockSpec((bs, D), lambda i: (i, 0)),
            pl.BlockSpec((bs, D), lambda i: (i, 0)),
            pl.BlockSpec((bs, D), lambda i: (i, 0)),
            pl.BlockSpec((bs, ED), lambda i: (i, 0)),
            pl.BlockSpec((bs, ED), lambda i: (i, 0)),
            pl.BlockSpec((bs, ED), lambda i: (i, 0)),
            pl.BlockSpec((bs, ED), lambda i: (i, 0)),
            pl.BlockSpec((bs, ED), lambda i: (i, 0)),
            full(D, D), full(D, D), full(1, D), full(D, D), full(1, D),
        ],
        out_specs=[
            pl.BlockSpec((bs, D), lambda i: (i, 0)),
            pl.BlockSpec((bs, ED), lambda i: (i, 0)),
        ],
        out_shape=[
            jax.ShapeDtypeStruct((N, D), _f32),
            jax.ShapeDtypeStruct((N, ED), _f32),
        ],
    )(x, *aggs, *cags, c16, wn1x, wn1a, bn1, wn2, bn2)


def kernel(x, edge_index, coords, edge_attr,
           We1, be1, We2, be2, Wc1, bc1, Wc2, Wn1, bn1, Wn2, bn2):
    row = edge_index[0].astype(_i32)
    col = edge_index[1].astype(_i32)
    rowg = row.reshape(NBT, TB)
    colg = col.reshape(NBT, TB)
    c16 = jnp.pad(coords, ((0, 0), (0, ED - 3)))
    eat = edge_attr.T

    xr1, xc1 = _node_pre(x, We1[:D], We1[D:2 * D])

    we1e = We1[2 * D:2 * D + ED]
    wd = We1[2 * D + ED:]
    z128 = jnp.zeros((RPT, D), _f32)
    z16 = jnp.zeros((RPT, ED), _f32)

    aggs, cags = [], []
    for boff, b0, b1 in CH:
        nbatch = NS * (b0 + b1)
        g1, g2, crc = _edge_gather(xr1, xc1, c16, rowg, colg, boff, b0, b1)
        ef, cu = _edge_mlp(g1, g2, crc, eat, we1e, wd, be1.reshape(1, D),
                           We2, be2.reshape(1, D), Wc1, bc1.reshape(1, D),
                           Wc2, boff * TB)
        aggs.append(_scatter(ef, rowg, z128, D, D, boff, nbatch))
        cags.append(_scatter(cu, rowg, z16, D, ED, boff, nbatch))

    a = [p[i, :N] for p in aggs for i in range(NC)]
    cg = [p[i, :N] for p in cags for i in range(NC)]
    xn, cn = _node_mlp(x, a, cg, c16, Wn1[:D], Wn1[D:], bn1.reshape(1, D),
                       Wn2, bn2.reshape(1, D))
    return (xn, cn[:, :3])
